# Initial kernel scaffold; baseline (speedup 1.0000x reference)
import jax
import jax.numpy as jnp
from jax.experimental import pallas as pl


def kernel(x, edge_index, edge_attr, return_attention_weights, params):
    raise NotImplementedError("write your pallas kernel here")


# speedup vs baseline: 1.1237x; 1.1237x over previous
import functools

import jax
import jax.numpy as jnp
from jax.experimental import pallas as pl
from jax.experimental.pallas import tpu as pltpu

N = 10000
E = 320000
H = 8
C = 64


def _lin_body(x_ref, w_ref, b_ref, o_ref):
    o_ref[...] = x_ref[...] @ w_ref[...] + b_ref[...]


def _linear_pallas(x, w, b):
    n, _ = x.shape
    _, m = w.shape
    return pl.pallas_call(
        _lin_body,
        out_shape=jax.ShapeDtypeStruct((n, m), jnp.float32),
    )(x, w, b.reshape(1, m))


def _layer(h_in, src, dst, ea, ea_mean, pp, concat):
    W = pp["W"]
    h = (h_in @ W).reshape(N, H, C)
    a_src = (h * pp["att_src"]).sum(-1)
    a_dst = (h * pp["att_dst"]).sum(-1)
    Ve = (pp["W_e"].reshape(64, H, C) * pp["att_e"][None]).sum(-1)
    a_e = ea @ Ve
    a_e_loop = ea_mean @ Ve
    B = (jnp.max(a_src, 0) + jnp.max(a_dst, 0)
         + jnp.maximum(jnp.max(a_e, 0), a_e_loop))
    Bl = jnp.where(B >= 0.0, B, 0.2 * B)

    logit = jax.nn.leaky_relu(a_src[src] + a_dst[dst] + a_e, 0.2)
    ex = jnp.exp(logit - Bl)
    den = jax.ops.segment_sum(ex, dst, num_segments=N)

    logit_l = jax.nn.leaky_relu(a_src + a_dst + a_e_loop, 0.2)
    ex_l = jnp.exp(logit_l - Bl)
    den = den + ex_l
    rec = 1.0 / (den + 1e-16)

    agg = jax.ops.segment_sum(h[src] * ex[..., None], dst, num_segments=N)
    agg = agg + h * ex_l[..., None]
    outm = agg * rec[..., None]
    alpha = ex * rec[dst]
    if concat:
        o = outm.reshape(N, H * C)
    else:
        o = outm.mean(axis=1)
    return o + pp["b"], alpha


def kernel(x, edge_index, edge_attr, return_attention_weights, params):
    p = params
    src = edge_index[0]
    dst = edge_index[1]
    out0 = x @ p["ne_W"] + p["ne_b"]
    skip = x @ p["skip_W"] + p["skip_b"]
    ea = edge_attr @ p["ee_W"] + p["ee_b"]
    ea_mean = ea.mean(axis=0)

    o1, _ = _layer(out0, src, dst, ea, ea_mean, p["c1"], True)
    o1 = jax.nn.elu(o1) + skip
    o2, _ = _layer(o1, src, dst, ea, ea_mean, p["c2"], True)
    o2 = jax.nn.elu(o2) + skip
    o3, alpha = _layer(o2, src, dst, ea, ea_mean, p["c3"], False)
    node_emb = jax.nn.elu(o3)

    x_out = _linear_pallas(node_emb, p["lin_W"], p["lin_b"])
    ee = jnp.concatenate([node_emb[src], alpha, node_emb[dst]], axis=-1)
    hid = jax.nn.relu(ee @ p["mlp_W1"] + p["mlp_b1"])
    edge_out = hid @ p["mlp_W2"] + p["mlp_b2"]
    return x_out, edge_out


# speedup vs baseline: 16.8556x; 15.0001x over previous
import functools

import jax
import jax.numpy as jnp
from jax import lax
from jax.experimental import pallas as pl
from jax.experimental.pallas import tpu as pltpu
from jax.experimental.pallas import tpu_sc as plsc

N = 10000
E = 320000
H = 8
C = 64
NP = 10240
ROWS = E // 128
RSC = ROWS // 2
BLK = 1280
NBLK = NP // BLK

_f32 = jnp.float32
_i32 = jnp.int32


def _ka_body(src_hbm, dst_hbm, ea_hbm, asrcT_hbm, adstT_hbm, ps_hbm,
             exT_hbm, den_hbm,
             asrc_v, adst_v, srcb, dstb, eab, exs, idxs, pbuf, zbuf, den_sp):
    c = lax.axis_index("c")
    s = lax.axis_index("s")

    @pl.loop(0, 64)
    def _(i):
        zbuf[pl.ds(i * 16, 16)] = jnp.zeros((16,), _f32)

    base0 = s * 5120
    for i in range(5):
        pltpu.sync_copy(zbuf, den_sp.at[pl.ds(base0 + i * 1024, 1024)])
    pltpu.sync_copy(ps_hbm, pbuf)
    plsc.subcore_barrier()

    for g in range(2):
        pltpu.sync_copy(asrcT_hbm.at[pl.ds(4 * g * NP, 4 * NP)], asrc_v)
        pltpu.sync_copy(adstT_hbm.at[pl.ds(4 * g * NP, 4 * NP)], adst_v)
        ms = [pbuf[0, 4 * g + h4] for h4 in range(4)]
        cs = [pbuf[1, 4 * g + h4] for h4 in range(4)]
        bs = [pbuf[2, 4 * g + h4] for h4 in range(4)]

        def chunk(row, g=g, ms=ms, cs=cs, bs=bs):
            base = row * 128
            pltpu.sync_copy(src_hbm.at[pl.ds(base, 128)], srcb)
            pltpu.sync_copy(dst_hbm.at[pl.ds(base, 128)], dstb)
            pltpu.sync_copy(ea_hbm.at[pl.ds(base, 128)], eab)

            @pl.loop(0, 8)
            def _(j):
                sv = srcb[pl.ds(j * 16, 16)]
                dv = dstb[pl.ds(j * 16, 16)]
                ev = eab[pl.ds(j * 16, 16)]
                for h4 in range(4):
                    h = 4 * g + h4
                    av = plsc.load_gather(asrc_v, [sv + h4 * NP])
                    bv = plsc.load_gather(adst_v, [dv + h4 * NP])
                    lg = av + bv + ms[h4] * ev + cs[h4]
                    lg = jnp.maximum(lg, 0.2 * lg)
                    exv = jnp.exp(lg - bs[h4])
                    exs[h4, pl.ds(j * 16, 16)] = exv
                    idxs[h4, pl.ds(j * 16, 16)] = dv + h * NP

            for h4 in range(4):
                h = 4 * g + h4
                pltpu.sync_copy(exs.at[h4], exT_hbm.at[h, pl.ds(base, 128)])
                pltpu.sync_copy(exs.at[h4], den_sp.at[idxs.at[h4]], add=True)

        tile_row0 = c * RSC + s * 78

        @pl.loop(0, 78)
        def _(k):
            chunk(tile_row0 + k)

        @pl.when(s < 2)
        def _():
            chunk(c * RSC + 1248 + s)

    plsc.subcore_barrier()
    pltpu.sync_copy(den_sp.at[pl.ds(s * 5120, 5120)],
                    den_hbm.at[c, pl.ds(s * 5120, 5120)])


def _kernel_a(src, dst, eattr, asrcT, adstT, psplat):
    return pl.kernel(
        _ka_body,
        out_type=[jax.ShapeDtypeStruct((H, E), _f32),
                  jax.ShapeDtypeStruct((2, 8 * NP), _f32)],
        mesh=plsc.VectorSubcoreMesh(core_axis_name="c", subcore_axis_name="s"),
        compiler_params=pltpu.CompilerParams(needs_layout_passes=False),
        scratch_types=[
            pltpu.VMEM((4 * NP,), _f32),
            pltpu.VMEM((4 * NP,), _f32),
            pltpu.VMEM((128,), _i32),
            pltpu.VMEM((128,), _i32),
            pltpu.VMEM((128,), _f32),
            pltpu.VMEM((4, 128), _f32),
            pltpu.VMEM((4, 128), _i32),
            pltpu.VMEM((3, 8, 16), _f32),
            pltpu.VMEM((1024,), _f32),
            pltpu.VMEM_SHARED((8 * NP,), _f32),
        ],
    )(src, dst, eattr, asrcT.reshape(8 * NP), adstT.reshape(8 * NP), psplat)


def _kb_body(src_hbm, dst_hbm, exT_hbm, hflat_hbm, zrow_hbm,
             agg_hbm,
             srcb, srcp, dstb, exa, exb, gbuf, acc_sp):
    c = lax.axis_index("c")
    s = lax.axis_index("s")

    for q in range(2):
        p = 2 * c + q
        for i in range(10):
            pltpu.sync_copy(zrow_hbm, acc_sp.at[pl.ds(s * 640 + i * 64, 64)])
        plsc.subcore_barrier()

        def chunk(row, p=p):
            base = row * 128
            pltpu.sync_copy(src_hbm.at[pl.ds(base, 128)], srcb)
            pltpu.sync_copy(dst_hbm.at[pl.ds(base, 128)], dstb)
            pltpu.sync_copy(exT_hbm.at[2 * p, pl.ds(base, 128)], exa)
            pltpu.sync_copy(exT_hbm.at[2 * p + 1, pl.ds(base, 128)], exb)
            off = p * NP
            for j in range(8):
                srcp[pl.ds(j * 16, 16)] = srcb[pl.ds(j * 16, 16)] + off
            pltpu.sync_copy(hflat_hbm.at[srcp], gbuf)

            @pl.loop(0, 8)
            def _(j16):
                exav = exa[pl.ds(j16 * 16, 16)]
                exbv = exb[pl.ds(j16 * 16, 16)]
                for jj in range(16):
                    jdx = j16 * 16 + jj
                    sa = exav[jj]
                    sb = exbv[jj]
                    for cc in range(4):
                        gbuf[jdx, pl.ds(cc * 16, 16)] = (
                            gbuf[jdx, pl.ds(cc * 16, 16)] * sa)
                    for cc in range(4, 8):
                        gbuf[jdx, pl.ds(cc * 16, 16)] = (
                            gbuf[jdx, pl.ds(cc * 16, 16)] * sb)

            pltpu.sync_copy(gbuf, acc_sp.at[dstb], add=True)

        row0 = s * 156

        @pl.loop(0, 156)
        def _(k):
            chunk(row0 + k)

        @pl.when(s < 4)
        def _():
            chunk(2496 + s)

        plsc.subcore_barrier()
        pltpu.sync_copy(acc_sp.at[pl.ds(s * 640, 640)],
                        agg_hbm.at[p, pl.ds(s * 640, 640)])
        plsc.subcore_barrier()


def _kernel_b(src, dst, exT, hflat, zrow):
    return pl.kernel(
        _kb_body,
        out_type=[jax.ShapeDtypeStruct((4, NP, 128), _f32)],
        mesh=plsc.VectorSubcoreMesh(core_axis_name="c", subcore_axis_name="s"),
        compiler_params=pltpu.CompilerParams(needs_layout_passes=False),
        scratch_types=[
            pltpu.VMEM((128,), _i32),
            pltpu.VMEM((128,), _i32),
            pltpu.VMEM((128,), _i32),
            pltpu.VMEM((128,), _f32),
            pltpu.VMEM((128,), _f32),
            pltpu.VMEM((128, 128), _f32),
            pltpu.VMEM_SHARED((NP, 128), _f32),
        ],
    )(src, dst, exT, hflat, zrow)[0]


def _kc_body(src_hbm, dst_hbm, emb_hbm, exT_hbm, rec_hbm,
             gsrc_hbm, gdst_hbm, alphaT_hbm,
             srcb, dstb, gsb, packb, exc, alpb, recv):
    c = lax.axis_index("c")
    s = lax.axis_index("s")
    pltpu.sync_copy(rec_hbm, recv)

    def chunk(row):
        base = row * 128
        pltpu.sync_copy(src_hbm.at[pl.ds(base, 128)], srcb)
        pltpu.sync_copy(dst_hbm.at[pl.ds(base, 128)], dstb)

        def gather_pack(idxb, out_hbm):
            pltpu.sync_copy(emb_hbm.at[idxb], gsb)

            @pl.loop(0, 8)
            def _(j16):
                for jj in range(16):
                    j = j16 * 16 + jj
                    for cc in range(4):
                        packb[pl.ds(j * 64 + cc * 16, 16)] = (
                            gsb[j, pl.ds(cc * 16, 16)])

            pltpu.sync_copy(packb, out_hbm.at[pl.ds(base * 64, 8192)])

        gather_pack(srcb, gsrc_hbm)
        gather_pack(dstb, gdst_hbm)

        for h in range(8):
            pltpu.sync_copy(exT_hbm.at[h, pl.ds(base, 128)], exc.at[h])

        @pl.loop(0, 8)
        def _(j):
            dv = dstb[pl.ds(j * 16, 16)]
            for h in range(8):
                rv = plsc.load_gather(recv, [dv + h * NP])
                ev = exc[h, pl.ds(j * 16, 16)]
                alpb[h, pl.ds(j * 16, 16)] = ev * rv

        for h in range(8):
            pltpu.sync_copy(alpb.at[h], alphaT_hbm.at[h, pl.ds(base, 128)])

    row0 = c * RSC + s * 78

    @pl.loop(0, 78)
    def _(k):
        chunk(row0 + k)

    @pl.when(s < 2)
    def _():
        chunk(c * RSC + 1248 + s)


def _kernel_c(src, dst, emb2, exT, recpl):
    return pl.kernel(
        _kc_body,
        out_type=[jax.ShapeDtypeStruct((E * 64,), _f32),
                  jax.ShapeDtypeStruct((E * 64,), _f32),
                  jax.ShapeDtypeStruct((H, E), _f32)],
        mesh=plsc.VectorSubcoreMesh(core_axis_name="c", subcore_axis_name="s"),
        compiler_params=pltpu.CompilerParams(needs_layout_passes=False),
        scratch_types=[
            pltpu.VMEM((128,), _i32),
            pltpu.VMEM((128,), _i32),
            pltpu.VMEM((128, 128), _f32),
            pltpu.VMEM((8192,), _f32),
            pltpu.VMEM((8, 128), _f32),
            pltpu.VMEM((8, 128), _f32),
            pltpu.VMEM((8 * NP,), _f32),
        ],
    )(src, dst, emb2, exT, recpl.reshape(8 * NP))


def _prep_tc(xp, neW, neb, skW, skb):
    def body(x_ref, nw_ref, nb_ref, sw_ref, sb_ref, z0_ref, skp_ref):
        xb = x_ref[...]
        p = pl.program_id(0)
        z0_ref[...] = jnp.dot(xb, nw_ref[...],
                              preferred_element_type=_f32) + nb_ref[...]
        skp_ref[0] = jnp.dot(xb, sw_ref[...],
                             preferred_element_type=_f32) + sb_ref[pl.ds(p, 1)]

    return pl.pallas_call(
        body,
        grid=(4, NBLK),
        in_specs=[pl.BlockSpec((BLK, 3), lambda p, i: (i, 0)),
                  pl.BlockSpec((3, 64), lambda p, i: (0, 0)),
                  pl.BlockSpec((1, 64), lambda p, i: (0, 0)),
                  pl.BlockSpec((3, 128), lambda p, i: (0, p)),
                  pl.BlockSpec((4, 128), lambda p, i: (0, 0))],
        out_specs=[pl.BlockSpec((BLK, 64), lambda p, i: (i, 0)),
                   pl.BlockSpec((1, BLK, 128), lambda p, i: (p, i, 0))],
        out_shape=[jax.ShapeDtypeStruct((NP, 64), _f32),
                   jax.ShapeDtypeStruct((4, NP, 128), _f32)],
    )(xp, neW, neb.reshape(1, 64), skW, skb.reshape(4, 128))


def _estats_tc(e2d):
    def body(e_ref, o_ref):
        v = e_ref[...]
        o_ref[0, :] = jnp.full((128,), jnp.min(v), _f32)
        o_ref[1, :] = jnp.full((128,), jnp.max(v), _f32)
        o_ref[2, :] = jnp.full((128,), jnp.mean(v), _f32)
        o_ref[3, :] = jnp.zeros((128,), _f32)

    return pl.pallas_call(
        body,
        out_shape=jax.ShapeDtypeStruct((4, 128), _f32),
    )(e2d)


def _asd1_tc(z0, ucat):
    def body(z_ref, u_ref, o_ref):
        o_ref[...] = jnp.dot(z_ref[...], u_ref[...],
                             preferred_element_type=_f32)

    return pl.pallas_call(
        body,
        grid=(NBLK,),
        in_specs=[pl.BlockSpec((BLK, 64), lambda i: (i, 0)),
                  pl.BlockSpec((64, 16), lambda i: (0, 0))],
        out_specs=pl.BlockSpec((BLK, 16), lambda i: (i, 0)),
        out_shape=jax.ShapeDtypeStruct((NP, 16), _f32),
    )(z0, ucat)


def _asd_tc(oP, ucat):
    def body(o_ref, u_ref, out_ref):
        part = jnp.dot(o_ref[0], u_ref[...], preferred_element_type=_f32)

        @pl.when(pl.program_id(1) == 0)
        def _():
            out_ref[...] = jnp.zeros_like(out_ref)

        out_ref[...] += part

    return pl.pallas_call(
        body,
        grid=(NBLK, 4),
        in_specs=[pl.BlockSpec((1, BLK, 128), lambda i, p: (p, i, 0)),
                  pl.BlockSpec((128, 16), lambda i, p: (p, 0))],
        out_specs=pl.BlockSpec((BLK, 16), lambda i, p: (i, 0)),
        out_shape=jax.ShapeDtypeStruct((NP, 16), _f32),
    )(oP, ucat)


def _pre1_tc(z0, w):
    def body(z_ref, w_ref, h_ref):
        h_ref[0] = jnp.dot(z_ref[...], w_ref[...],
                           preferred_element_type=_f32)

    return pl.pallas_call(
        body,
        grid=(4, NBLK),
        in_specs=[pl.BlockSpec((BLK, 64), lambda q, i: (i, 0)),
                  pl.BlockSpec((64, 128), lambda q, i: (0, q))],
        out_specs=pl.BlockSpec((1, BLK, 128), lambda q, i: (q, i, 0)),
        out_shape=jax.ShapeDtypeStruct((4, NP, 128), _f32),
    )(z0, w)


def _pre_tc(oP, w):
    def body(o_ref, w_ref, h_ref):
        part = jnp.dot(o_ref[0], w_ref[...], preferred_element_type=_f32)

        @pl.when(pl.program_id(2) == 0)
        def _():
            h_ref[...] = jnp.zeros_like(h_ref)

        h_ref[0] += part

    return pl.pallas_call(
        body,
        grid=(4, NBLK, 4),
        in_specs=[pl.BlockSpec((1, BLK, 128), lambda q, i, p: (p, i, 0)),
                  pl.BlockSpec((128, 128), lambda q, i, p: (p, q))],
        out_specs=pl.BlockSpec((1, BLK, 128), lambda q, i, p: (q, i, 0)),
        out_shape=jax.ShapeDtypeStruct((4, NP, 128), _f32),
    )(oP, w)


def _dens_tc(ascols, denC, consts):
    def body(a_ref, d_ref, c_ref, rec_ref, exl_ref):
        a = a_ref[...]
        l = a[:, :8] + a[:, 8:] + c_ref[0:1, :]
        lr = jnp.maximum(l, 0.2 * l)
        exl = jnp.exp(lr - c_ref[1:2, :])
        exl_ref[...] = exl
        rec_ref[...] = 1.0 / (d_ref[...] + exl + 1e-16)

    return pl.pallas_call(
        body,
        grid=(NBLK,),
        in_specs=[pl.BlockSpec((BLK, 16), lambda i: (i, 0)),
                  pl.BlockSpec((BLK, 8), lambda i: (i, 0)),
                  pl.BlockSpec((2, 8), lambda i: (0, 0))],
        out_specs=[pl.BlockSpec((BLK, 8), lambda i: (i, 0)),
                   pl.BlockSpec((BLK, 8), lambda i: (i, 0))],
        out_shape=[jax.ShapeDtypeStruct((NP, 8), _f32),
                   jax.ShapeDtypeStruct((NP, 8), _f32)],
    )(ascols, denC, consts)


def _post_tc(aggP, hP, recT, exlT, skipP, bP):
    def body(g_ref, h_ref, r_ref, x_ref, s_ref, b_ref, o_ref):
        for p in range(4):
            rec2 = r_ref[:, 2 * p:2 * p + 2]
            exl2 = x_ref[:, 2 * p:2 * p + 2]
            recs = jnp.concatenate(
                [jnp.broadcast_to(rec2[:, 0:1], (BLK, 64)),
                 jnp.broadcast_to(rec2[:, 1:2], (BLK, 64))], axis=1)
            exls = jnp.concatenate(
                [jnp.broadcast_to(exl2[:, 0:1], (BLK, 64)),
                 jnp.broadcast_to(exl2[:, 1:2], (BLK, 64))], axis=1)
            v = (g_ref[p] + h_ref[p] * exls) * recs + b_ref[p:p + 1]
            o_ref[p] = jnp.where(v > 0, v, (jnp.exp(v) - 1.0)) + s_ref[p]

    return pl.pallas_call(
        body,
        grid=(NBLK,),
        in_specs=[pl.BlockSpec((4, BLK, 128), lambda i: (0, i, 0)),
                  pl.BlockSpec((4, BLK, 128), lambda i: (0, i, 0)),
                  pl.BlockSpec((BLK, 8), lambda i: (i, 0)),
                  pl.BlockSpec((BLK, 8), lambda i: (i, 0)),
                  pl.BlockSpec((4, BLK, 128), lambda i: (0, i, 0)),
                  pl.BlockSpec((4, 128), lambda i: (0, 0))],
        out_specs=pl.BlockSpec((4, BLK, 128), lambda i: (0, i, 0)),
        out_shape=jax.ShapeDtypeStruct((4, NP, 128), _f32),
    )(aggP, hP, recT, exlT, skipP, bP)


def _post3_tc(aggP, hP, recT, exlT, b3, linW, linb):
    def body(g_ref, h_ref, r_ref, x_ref, b_ref, lw_ref, lb_ref,
             emb_ref, xo_ref):
        ssum = jnp.zeros((BLK, 64), _f32)
        for p in range(4):
            for half in range(2):
                hh = 2 * p + half
                seg = g_ref[p, :, 64 * half:64 * half + 64]
                hseg = h_ref[p, :, 64 * half:64 * half + 64]
                ssum += (seg + hseg * x_ref[:, hh:hh + 1]) * r_ref[:, hh:hh + 1]
        o = ssum * 0.125 + b_ref[...]
        embv = jnp.where(o > 0, o, (jnp.exp(o) - 1.0))
        emb_ref[...] = jnp.concatenate(
            [embv, jnp.zeros((BLK, 64), _f32)], axis=1)
        xo_ref[...] = jnp.dot(embv, lw_ref[...],
                              preferred_element_type=_f32) + lb_ref[...]

    return pl.pallas_call(
        body,
        grid=(NBLK,),
        in_specs=[pl.BlockSpec((4, BLK, 128), lambda i: (0, i, 0)),
                  pl.BlockSpec((4, BLK, 128), lambda i: (0, i, 0)),
                  pl.BlockSpec((BLK, 8), lambda i: (i, 0)),
                  pl.BlockSpec((BLK, 8), lambda i: (i, 0)),
                  pl.BlockSpec((1, 64), lambda i: (0, 0)),
                  pl.BlockSpec((64, 2), lambda i: (0, 0)),
                  pl.BlockSpec((1, 2), lambda i: (0, 0))],
        out_specs=[pl.BlockSpec((BLK, 128), lambda i: (i, 0)),
                   pl.BlockSpec((BLK, 2), lambda i: (i, 0))],
        out_shape=[jax.ShapeDtypeStruct((NP, 128), _f32),
                   jax.ShapeDtypeStruct((NP, 2), _f32)],
    )(aggP, hP, recT, exlT, b3, linW, linb)


def _mlp_tc(gsrc, gdst, alpha, a1, a2, a3, b1, w2, b2):
    eblk = 2000

    def body(g1_ref, g2_ref, al_ref, a1_ref, a2_ref, a3_ref, b1_ref,
             w2_ref, b2_ref, o_ref):
        hid = (jnp.dot(g1_ref[...], a1_ref[...], preferred_element_type=_f32)
               + jnp.dot(al_ref[...], a2_ref[...], preferred_element_type=_f32)
               + jnp.dot(g2_ref[...], a3_ref[...], preferred_element_type=_f32)
               + b1_ref[...])
        hid = jnp.maximum(hid, 0.0)
        o_ref[...] = jnp.dot(hid, w2_ref[...],
                             preferred_element_type=_f32) + b2_ref[...]

    return pl.pallas_call(
        body,
        grid=(E // eblk,),
        in_specs=[pl.BlockSpec((eblk, 64), lambda i: (i, 0)),
                  pl.BlockSpec((eblk, 64), lambda i: (i, 0)),
                  pl.BlockSpec((eblk, 8), lambda i: (i, 0)),
                  pl.BlockSpec((64, 256), lambda i: (0, 0)),
                  pl.BlockSpec((8, 256), lambda i: (0, 0)),
                  pl.BlockSpec((64, 256), lambda i: (0, 0)),
                  pl.BlockSpec((1, 256), lambda i: (0, 0)),
                  pl.BlockSpec((256, 2), lambda i: (0, 0)),
                  pl.BlockSpec((1, 2), lambda i: (0, 0))],
        out_specs=pl.BlockSpec((eblk, 2), lambda i: (i, 0)),
        out_shape=jax.ShapeDtypeStruct((E, 2), _f32),
    )(gsrc, gdst, alpha, a1, a2, a3, b1, w2, b2)


def kernel(x, edge_index, edge_attr, return_attention_weights, params):
    p = params
    src = edge_index[0]
    dst = edge_index[1]
    eattr = edge_attr[:, 0]
    xp = jnp.pad(x, ((0, NP - N), (0, 0)))
    zrow = jnp.zeros((64, 128), _f32)

    est = _estats_tc(eattr.reshape(ROWS, 128))
    emin, emax, emean = est[0, 0], est[1, 0], est[2, 0]

    z0, skipP = _prep_tc(xp, p["ne_W"], p["ne_b"], p["skip_W"], p["skip_b"])

    def layer(feats, pp, first):
        w = pp["W"]
        ind = w.shape[0]
        us = (w.reshape(ind, H, C) * pp["att_src"][None]).sum(-1)
        ud = (w.reshape(ind, H, C) * pp["att_dst"][None]).sum(-1)
        ucat = jnp.concatenate([us, ud], axis=1)
        ve = (pp["W_e"].reshape(64, H, C) * pp["att_e"][None]).sum(-1)
        mvec = (p["ee_W"] @ ve)[0]
        cvec = p["ee_b"] @ ve
        ael = emean * mvec + cvec

        if first:
            hP = _pre1_tc(feats, w)
            ascols = _asd1_tc(feats, ucat)
        else:
            hP = _pre_tc(feats, w)
            ascols = _asd_tc(feats, ucat)

        asrc = ascols[:N, :8]
        adst = ascols[:N, 8:]
        ae_max = jnp.maximum(jnp.where(mvec > 0, mvec * emax, mvec * emin)
                             + cvec, ael)
        b = jnp.max(asrc, axis=0) + jnp.max(adst, axis=0) + ae_max
        bl = jnp.where(b >= 0.0, b, 0.2 * b)

        psplat = jnp.stack([
            jnp.broadcast_to(mvec[:, None], (8, 16)),
            jnp.broadcast_to(cvec[:, None], (8, 16)),
            jnp.broadcast_to(bl[:, None], (8, 16))])

        asrcT = jnp.copy(ascols[:, :8].T)
        adstT = jnp.copy(ascols[:, 8:].T)
        exT, den2 = _kernel_a(src, dst, eattr, asrcT, adstT, psplat)

        den = (den2[0] + den2[1]).reshape(8, NP)
        denC = jnp.copy(den.T)
        consts = jnp.stack([ael, bl])
        recT, exlT = _dens_tc(ascols, denC, consts)

        hflat = hP.reshape(4 * NP, 128)
        aggP = _kernel_b(src, dst, exT, hflat, zrow)
        return hP, aggP, recT, exlT, exT

    hP, aggP, recT, exlT, _ = layer(z0, p["c1"], True)
    b1p = p["c1"]["b"].reshape(4, 128)
    oP = _post_tc(aggP, hP, recT, exlT, skipP, b1p)

    hP, aggP, recT, exlT, _ = layer(oP, p["c2"], False)
    b2p = p["c2"]["b"].reshape(4, 128)
    oP = _post_tc(aggP, hP, recT, exlT, skipP, b2p)

    hP, aggP, recT, exlT, exT3 = layer(oP, p["c3"], False)
    emb, xoutp = _post3_tc(aggP, hP, recT, exlT,
                           p["c3"]["b"].reshape(1, 64), p["lin_W"],
                           p["lin_b"].reshape(1, 2))

    rec3pl = jnp.copy(recT.T)
    gsrcf, gdstf, alphaT = _kernel_c(src, dst, emb, exT3, rec3pl)
    gsrc = gsrcf.reshape(E, 64)
    gdst = gdstf.reshape(E, 64)
    alpha = alphaT.T

    w1 = p["mlp_W1"]
    edge_out = _mlp_tc(gsrc, gdst, alpha,
                       w1[:64], w1[64:72], w1[72:],
                       p["mlp_b1"].reshape(1, 256), p["mlp_W2"],
                       p["mlp_b2"].reshape(1, 2))
    return xoutp[:N], edge_out


# speedup vs baseline: 21.1199x; 1.2530x over previous
import functools

import jax
import jax.numpy as jnp
from jax import lax
from jax.experimental import pallas as pl
from jax.experimental.pallas import tpu as pltpu
from jax.experimental.pallas import tpu_sc as plsc

N = 10000
E = 320000
H = 8
C = 64
NP = 10240
ROWS = E // 128
RSC = ROWS // 2
BLK = 1280
NBLK = NP // BLK

_f32 = jnp.float32
_i32 = jnp.int32


def _ka_body(src_hbm, dst_hbm, ea_hbm, asrcT_hbm, adstT_hbm, ps_hbm,
             exT_hbm, den_hbm,
             asrc_v, adst_v, srcb, dstb, eab, exs, idxs, pbuf, zbuf, den_sp):
    c = lax.axis_index("c")
    s = lax.axis_index("s")

    @pl.loop(0, 64)
    def _(i):
        zbuf[pl.ds(i * 16, 16)] = jnp.zeros((16,), _f32)

    base0 = s * 5120
    for i in range(5):
        pltpu.sync_copy(zbuf, den_sp.at[pl.ds(base0 + i * 1024, 1024)])
    pltpu.sync_copy(ps_hbm, pbuf)
    plsc.subcore_barrier()

    for g in range(2):
        pltpu.sync_copy(asrcT_hbm.at[pl.ds(4 * g * NP, 4 * NP)], asrc_v)
        pltpu.sync_copy(adstT_hbm.at[pl.ds(4 * g * NP, 4 * NP)], adst_v)
        ms = [pbuf[0, 4 * g + h4] for h4 in range(4)]
        cs = [pbuf[1, 4 * g + h4] for h4 in range(4)]
        bs = [pbuf[2, 4 * g + h4] for h4 in range(4)]

        def chunk(row, g=g, ms=ms, cs=cs, bs=bs):
            base = row * 128
            pltpu.sync_copy(src_hbm.at[pl.ds(base, 128)], srcb)
            pltpu.sync_copy(dst_hbm.at[pl.ds(base, 128)], dstb)
            pltpu.sync_copy(ea_hbm.at[pl.ds(base, 128)], eab)

            @pl.loop(0, 8)
            def _(j):
                sv = srcb[pl.ds(j * 16, 16)]
                dv = dstb[pl.ds(j * 16, 16)]
                ev = eab[pl.ds(j * 16, 16)]
                for h4 in range(4):
                    h = 4 * g + h4
                    av = plsc.load_gather(asrc_v, [sv + h4 * NP])
                    bv = plsc.load_gather(adst_v, [dv + h4 * NP])
                    lg = av + bv + ms[h4] * ev + cs[h4]
                    lg = jnp.maximum(lg, 0.2 * lg)
                    exv = jnp.exp(lg - bs[h4])
                    exs[h4, pl.ds(j * 16, 16)] = exv
                    idxs[h4, pl.ds(j * 16, 16)] = dv + h * NP

            for h4 in range(4):
                h = 4 * g + h4
                pltpu.sync_copy(exs.at[h4], exT_hbm.at[h, pl.ds(base, 128)])
                pltpu.sync_copy(exs.at[h4], den_sp.at[idxs.at[h4]], add=True)

        tile_row0 = c * RSC + s * 78

        @pl.loop(0, 78)
        def _(k):
            chunk(tile_row0 + k)

        @pl.when(s < 2)
        def _():
            chunk(c * RSC + 1248 + s)

    plsc.subcore_barrier()
    pltpu.sync_copy(den_sp.at[pl.ds(s * 5120, 5120)],
                    den_hbm.at[c, pl.ds(s * 5120, 5120)])


def _kernel_a(src, dst, eattr, asrcT, adstT, psplat):
    return pl.kernel(
        _ka_body,
        out_type=[jax.ShapeDtypeStruct((H, E), _f32),
                  jax.ShapeDtypeStruct((2, 8 * NP), _f32)],
        mesh=plsc.VectorSubcoreMesh(core_axis_name="c", subcore_axis_name="s"),
        compiler_params=pltpu.CompilerParams(needs_layout_passes=False),
        scratch_types=[
            pltpu.VMEM((4 * NP,), _f32),
            pltpu.VMEM((4 * NP,), _f32),
            pltpu.VMEM((128,), _i32),
            pltpu.VMEM((128,), _i32),
            pltpu.VMEM((128,), _f32),
            pltpu.VMEM((4, 128), _f32),
            pltpu.VMEM((4, 128), _i32),
            pltpu.VMEM((3, 8, 16), _f32),
            pltpu.VMEM((1024,), _f32),
            pltpu.VMEM_SHARED((8 * NP,), _f32),
        ],
    )(src, dst, eattr, asrcT.reshape(8 * NP), adstT.reshape(8 * NP), psplat)


def _kb_body(src_hbm, dst_hbm, exT_hbm, hflat_hbm, zrow_hbm,
             agg_hbm,
             srcb, dstb, exa, exb, srcp2, dstb2,
             g0, g1, sg0, sg1, ss0, ss1, acc_sp):
    c = lax.axis_index("c")
    s = lax.axis_index("s")
    gbufs = (g0, g1)
    sgs = (sg0, sg1)
    sss = (ss0, ss1)
    NR = 2

    for q in range(2):
        p = 2 * c + q
        for i in range(10):
            pltpu.sync_copy(zrow_hbm, acc_sp.at[pl.ds(s * 640 + i * 64, 64)])
        plsc.subcore_barrier()
        off = p * NP
        row0 = s * 156

        def scale(gb, exav, exbv, jdx0):
            for jj in range(16):
                jdx = jdx0 + jj
                sa = exav[jj]
                sb = exbv[jj]
                for cc in range(4):
                    gb[jdx, pl.ds(cc * 16, 16)] = (
                        gb[jdx, pl.ds(cc * 16, 16)] * sa)
                for cc in range(4, 8):
                    gb[jdx, pl.ds(cc * 16, 16)] = (
                        gb[jdx, pl.ds(cc * 16, 16)] * sb)

        @pl.loop(0, 78)
        def _(sk):
            r0 = row0 + sk * NR
            base = r0 * 128
            pltpu.sync_copy(src_hbm.at[pl.ds(base, 256)], srcb)
            pltpu.sync_copy(dst_hbm.at[pl.ds(base, 256)], dstb)
            pltpu.sync_copy(exT_hbm.at[2 * p, pl.ds(base, 256)], exa)
            pltpu.sync_copy(exT_hbm.at[2 * p + 1, pl.ds(base, 256)], exb)

            @pl.when(sk > 0)
            def _():
                for r in range(NR):
                    pltpu.make_async_copy(
                        gbufs[r], acc_sp.at[dstb2.at[r]], sss[r]).wait()

            for r in range(NR):
                for j in range(8):
                    srcp2[r, pl.ds(j * 16, 16)] = (
                        srcb[pl.ds(r * 128 + j * 16, 16)] + off)
                    dstb2[r, pl.ds(j * 16, 16)] = (
                        dstb[pl.ds(r * 128 + j * 16, 16)])
            for r in range(NR):
                pltpu.async_copy(hflat_hbm.at[srcp2.at[r]], gbufs[r], sgs[r])
            for r in range(NR):
                pltpu.make_async_copy(
                    hflat_hbm.at[srcp2.at[r]], gbufs[r], sgs[r]).wait()

                @pl.loop(0, 8)
                def _(j16, r=r):
                    exav = exa[pl.ds(r * 128 + j16 * 16, 16)]
                    exbv = exb[pl.ds(r * 128 + j16 * 16, 16)]
                    scale(gbufs[r], exav, exbv, j16 * 16)

                pltpu.async_copy(gbufs[r], acc_sp.at[dstb2.at[r]], sss[r],
                                 add=True)

        for r in range(NR):
            pltpu.make_async_copy(
                gbufs[r], acc_sp.at[dstb2.at[r]], sss[r]).wait()

        @pl.when(s < 4)
        def _():
            base = (2496 + s) * 128
            pltpu.sync_copy(src_hbm.at[pl.ds(base, 128)],
                            srcb.at[pl.ds(0, 128)])
            pltpu.sync_copy(dst_hbm.at[pl.ds(base, 128)],
                            dstb.at[pl.ds(0, 128)])
            pltpu.sync_copy(exT_hbm.at[2 * p, pl.ds(base, 128)],
                            exa.at[pl.ds(0, 128)])
            pltpu.sync_copy(exT_hbm.at[2 * p + 1, pl.ds(base, 128)],
                            exb.at[pl.ds(0, 128)])
            for j in range(8):
                srcp2[0, pl.ds(j * 16, 16)] = srcb[pl.ds(j * 16, 16)] + off
                dstb2[0, pl.ds(j * 16, 16)] = dstb[pl.ds(j * 16, 16)]
            pltpu.sync_copy(hflat_hbm.at[srcp2.at[0]], g0)

            @pl.loop(0, 8)
            def _(j16):
                exav = exa[pl.ds(j16 * 16, 16)]
                exbv = exb[pl.ds(j16 * 16, 16)]
                scale(g0, exav, exbv, j16 * 16)

            pltpu.sync_copy(g0, acc_sp.at[dstb2.at[0]], add=True)

        plsc.subcore_barrier()
        pltpu.sync_copy(acc_sp.at[pl.ds(s * 640, 640)],
                        agg_hbm.at[p, pl.ds(s * 640, 640)])
        plsc.subcore_barrier()


def _kernel_b(src, dst, exT, hflat, zrow):
    return pl.kernel(
        _kb_body,
        out_type=[jax.ShapeDtypeStruct((4, NP, 128), _f32)],
        mesh=plsc.VectorSubcoreMesh(core_axis_name="c", subcore_axis_name="s"),
        compiler_params=pltpu.CompilerParams(needs_layout_passes=False),
        scratch_types=(
            [pltpu.VMEM((256,), _i32),
             pltpu.VMEM((256,), _i32),
             pltpu.VMEM((256,), _f32),
             pltpu.VMEM((256,), _f32),
             pltpu.VMEM((2, 128), _i32),
             pltpu.VMEM((2, 128), _i32)]
            + [pltpu.VMEM((128, 128), _f32) for _ in range(2)]
            + [pltpu.SemaphoreType.DMA for _ in range(4)]
            + [pltpu.VMEM_SHARED((NP, 128), _f32)]),
    )(src, dst, exT, hflat, zrow)[0]


def _kc_body(src_hbm, dst_hbm, emb_hbm, exT_hbm, rec_hbm,
             gsrc_hbm, gdst_hbm, alphaT_hbm,
             srcb, dstb, gsb, packb, exc, alpb, recv):
    c = lax.axis_index("c")
    s = lax.axis_index("s")
    pltpu.sync_copy(rec_hbm, recv)

    def chunk(row):
        base = row * 128
        pltpu.sync_copy(src_hbm.at[pl.ds(base, 128)], srcb)
        pltpu.sync_copy(dst_hbm.at[pl.ds(base, 128)], dstb)

        def gather_pack(idxb, out_hbm):
            pltpu.sync_copy(emb_hbm.at[idxb], gsb)

            @pl.loop(0, 8)
            def _(j16):
                for jj in range(16):
                    j = j16 * 16 + jj
                    for cc in range(4):
                        packb[pl.ds(j * 64 + cc * 16, 16)] = (
                            gsb[j, pl.ds(cc * 16, 16)])

            pltpu.sync_copy(packb, out_hbm.at[pl.ds(base * 64, 8192)])

        gather_pack(srcb, gsrc_hbm)
        gather_pack(dstb, gdst_hbm)

        for h in range(8):
            pltpu.sync_copy(exT_hbm.at[h, pl.ds(base, 128)], exc.at[h])

        @pl.loop(0, 8)
        def _(j):
            dv = dstb[pl.ds(j * 16, 16)]
            for h in range(8):
                rv = plsc.load_gather(recv, [dv + h * NP])
                ev = exc[h, pl.ds(j * 16, 16)]
                alpb[h, pl.ds(j * 16, 16)] = ev * rv

        for h in range(8):
            pltpu.sync_copy(alpb.at[h], alphaT_hbm.at[h, pl.ds(base, 128)])

    row0 = c * RSC + s * 78

    @pl.loop(0, 78)
    def _(k):
        chunk(row0 + k)

    @pl.when(s < 2)
    def _():
        chunk(c * RSC + 1248 + s)


def _kernel_c(src, dst, emb2, exT, recpl):
    return pl.kernel(
        _kc_body,
        out_type=[jax.ShapeDtypeStruct((E * 64,), _f32),
                  jax.ShapeDtypeStruct((E * 64,), _f32),
                  jax.ShapeDtypeStruct((H, E), _f32)],
        mesh=plsc.VectorSubcoreMesh(core_axis_name="c", subcore_axis_name="s"),
        compiler_params=pltpu.CompilerParams(needs_layout_passes=False),
        scratch_types=[
            pltpu.VMEM((128,), _i32),
            pltpu.VMEM((128,), _i32),
            pltpu.VMEM((128, 128), _f32),
            pltpu.VMEM((8192,), _f32),
            pltpu.VMEM((8, 128), _f32),
            pltpu.VMEM((8, 128), _f32),
            pltpu.VMEM((8 * NP,), _f32),
        ],
    )(src, dst, emb2, exT, recpl.reshape(8 * NP))


def _prep_tc(xp, neW, neb, skW, skb):
    def body(x_ref, nw_ref, nb_ref, sw_ref, sb_ref, z0_ref, skp_ref):
        xb = x_ref[...]
        p = pl.program_id(0)
        z0_ref[...] = jnp.dot(xb, nw_ref[...],
                              preferred_element_type=_f32) + nb_ref[...]
        skp_ref[0] = jnp.dot(xb, sw_ref[...],
                             preferred_element_type=_f32) + sb_ref[pl.ds(p, 1)]

    return pl.pallas_call(
        body,
        grid=(4, NBLK),
        in_specs=[pl.BlockSpec((BLK, 3), lambda p, i: (i, 0)),
                  pl.BlockSpec((3, 64), lambda p, i: (0, 0)),
                  pl.BlockSpec((1, 64), lambda p, i: (0, 0)),
                  pl.BlockSpec((3, 128), lambda p, i: (0, p)),
                  pl.BlockSpec((4, 128), lambda p, i: (0, 0))],
        out_specs=[pl.BlockSpec((BLK, 64), lambda p, i: (i, 0)),
                   pl.BlockSpec((1, BLK, 128), lambda p, i: (p, i, 0))],
        out_shape=[jax.ShapeDtypeStruct((NP, 64), _f32),
                   jax.ShapeDtypeStruct((4, NP, 128), _f32)],
    )(xp, neW, neb.reshape(1, 64), skW, skb.reshape(4, 128))


def _estats_tc(e2d):
    def body(e_ref, o_ref):
        v = e_ref[...]
        o_ref[0, :] = jnp.full((128,), jnp.min(v), _f32)
        o_ref[1, :] = jnp.full((128,), jnp.max(v), _f32)
        o_ref[2, :] = jnp.full((128,), jnp.mean(v), _f32)
        o_ref[3, :] = jnp.zeros((128,), _f32)

    return pl.pallas_call(
        body,
        out_shape=jax.ShapeDtypeStruct((4, 128), _f32),
    )(e2d)


def _asd1_tc(z0, ucat):
    def body(z_ref, u_ref, o_ref):
        o_ref[...] = jnp.dot(z_ref[...], u_ref[...],
                             preferred_element_type=_f32)

    return pl.pallas_call(
        body,
        grid=(NBLK,),
        in_specs=[pl.BlockSpec((BLK, 64), lambda i: (i, 0)),
                  pl.BlockSpec((64, 16), lambda i: (0, 0))],
        out_specs=pl.BlockSpec((BLK, 16), lambda i: (i, 0)),
        out_shape=jax.ShapeDtypeStruct((NP, 16), _f32),
    )(z0, ucat)


def _asd_tc(oP, ucat):
    def body(o_ref, u_ref, out_ref):
        part = jnp.dot(o_ref[0], u_ref[...], preferred_element_type=_f32)

        @pl.when(pl.program_id(1) == 0)
        def _():
            out_ref[...] = jnp.zeros_like(out_ref)

        out_ref[...] += part

    return pl.pallas_call(
        body,
        grid=(NBLK, 4),
        in_specs=[pl.BlockSpec((1, BLK, 128), lambda i, p: (p, i, 0)),
                  pl.BlockSpec((128, 16), lambda i, p: (p, 0))],
        out_specs=pl.BlockSpec((BLK, 16), lambda i, p: (i, 0)),
        out_shape=jax.ShapeDtypeStruct((NP, 16), _f32),
    )(oP, ucat)


def _pre1_tc(z0, w):
    def body(z_ref, w_ref, h_ref):
        h_ref[0] = jnp.dot(z_ref[...], w_ref[...],
                           preferred_element_type=_f32)

    return pl.pallas_call(
        body,
        grid=(4, NBLK),
        in_specs=[pl.BlockSpec((BLK, 64), lambda q, i: (i, 0)),
                  pl.BlockSpec((64, 128), lambda q, i: (0, q))],
        out_specs=pl.BlockSpec((1, BLK, 128), lambda q, i: (q, i, 0)),
        out_shape=jax.ShapeDtypeStruct((4, NP, 128), _f32),
    )(z0, w)


def _pre_tc(oP, w):
    def body(o_ref, w_ref, h_ref):
        part = jnp.dot(o_ref[0], w_ref[...], preferred_element_type=_f32)

        @pl.when(pl.program_id(2) == 0)
        def _():
            h_ref[...] = jnp.zeros_like(h_ref)

        h_ref[0] += part

    return pl.pallas_call(
        body,
        grid=(4, NBLK, 4),
        in_specs=[pl.BlockSpec((1, BLK, 128), lambda q, i, p: (p, i, 0)),
                  pl.BlockSpec((128, 128), lambda q, i, p: (p, q))],
        out_specs=pl.BlockSpec((1, BLK, 128), lambda q, i, p: (q, i, 0)),
        out_shape=jax.ShapeDtypeStruct((4, NP, 128), _f32),
    )(oP, w)


def _dens_tc(ascols, denC, consts):
    def body(a_ref, d_ref, c_ref, rec_ref, exl_ref):
        a = a_ref[...]
        l = a[:, :8] + a[:, 8:] + c_ref[0:1, :]
        lr = jnp.maximum(l, 0.2 * l)
        exl = jnp.exp(lr - c_ref[1:2, :])
        exl_ref[...] = exl
        rec_ref[...] = 1.0 / (d_ref[...] + exl + 1e-16)

    return pl.pallas_call(
        body,
        grid=(NBLK,),
        in_specs=[pl.BlockSpec((BLK, 16), lambda i: (i, 0)),
                  pl.BlockSpec((BLK, 8), lambda i: (i, 0)),
                  pl.BlockSpec((2, 8), lambda i: (0, 0))],
        out_specs=[pl.BlockSpec((BLK, 8), lambda i: (i, 0)),
                   pl.BlockSpec((BLK, 8), lambda i: (i, 0))],
        out_shape=[jax.ShapeDtypeStruct((NP, 8), _f32),
                   jax.ShapeDtypeStruct((NP, 8), _f32)],
    )(ascols, denC, consts)


def _post_tc(aggP, hP, recT, exlT, skipP, bP):
    def body(g_ref, h_ref, r_ref, x_ref, s_ref, b_ref, o_ref):
        for p in range(4):
            rec2 = r_ref[:, 2 * p:2 * p + 2]
            exl2 = x_ref[:, 2 * p:2 * p + 2]
            recs = jnp.concatenate(
                [jnp.broadcast_to(rec2[:, 0:1], (BLK, 64)),
                 jnp.broadcast_to(rec2[:, 1:2], (BLK, 64))], axis=1)
            exls = jnp.concatenate(
                [jnp.broadcast_to(exl2[:, 0:1], (BLK, 64)),
                 jnp.broadcast_to(exl2[:, 1:2], (BLK, 64))], axis=1)
            v = (g_ref[p] + h_ref[p] * exls) * recs + b_ref[p:p + 1]
            o_ref[p] = jnp.where(v > 0, v, (jnp.exp(v) - 1.0)) + s_ref[p]

    return pl.pallas_call(
        body,
        grid=(NBLK,),
        in_specs=[pl.BlockSpec((4, BLK, 128), lambda i: (0, i, 0)),
                  pl.BlockSpec((4, BLK, 128), lambda i: (0, i, 0)),
                  pl.BlockSpec((BLK, 8), lambda i: (i, 0)),
                  pl.BlockSpec((BLK, 8), lambda i: (i, 0)),
                  pl.BlockSpec((4, BLK, 128), lambda i: (0, i, 0)),
                  pl.BlockSpec((4, 128), lambda i: (0, 0))],
        out_specs=pl.BlockSpec((4, BLK, 128), lambda i: (0, i, 0)),
        out_shape=jax.ShapeDtypeStruct((4, NP, 128), _f32),
    )(aggP, hP, recT, exlT, skipP, bP)


def _post3_tc(aggP, hP, recT, exlT, b3, linW, linb):
    def body(g_ref, h_ref, r_ref, x_ref, b_ref, lw_ref, lb_ref,
             emb_ref, xo_ref):
        ssum = jnp.zeros((BLK, 64), _f32)
        for p in range(4):
            for half in range(2):
                hh = 2 * p + half
                seg = g_ref[p, :, 64 * half:64 * half + 64]
                hseg = h_ref[p, :, 64 * half:64 * half + 64]
                ssum += (seg + hseg * x_ref[:, hh:hh + 1]) * r_ref[:, hh:hh + 1]
        o = ssum * 0.125 + b_ref[...]
        embv = jnp.where(o > 0, o, (jnp.exp(o) - 1.0))
        emb_ref[...] = jnp.concatenate(
            [embv, jnp.zeros((BLK, 64), _f32)], axis=1)
        xo_ref[...] = jnp.dot(embv, lw_ref[...],
                              preferred_element_type=_f32) + lb_ref[...]

    return pl.pallas_call(
        body,
        grid=(NBLK,),
        in_specs=[pl.BlockSpec((4, BLK, 128), lambda i: (0, i, 0)),
                  pl.BlockSpec((4, BLK, 128), lambda i: (0, i, 0)),
                  pl.BlockSpec((BLK, 8), lambda i: (i, 0)),
                  pl.BlockSpec((BLK, 8), lambda i: (i, 0)),
                  pl.BlockSpec((1, 64), lambda i: (0, 0)),
                  pl.BlockSpec((64, 2), lambda i: (0, 0)),
                  pl.BlockSpec((1, 2), lambda i: (0, 0))],
        out_specs=[pl.BlockSpec((BLK, 128), lambda i: (i, 0)),
                   pl.BlockSpec((BLK, 2), lambda i: (i, 0))],
        out_shape=[jax.ShapeDtypeStruct((NP, 128), _f32),
                   jax.ShapeDtypeStruct((NP, 2), _f32)],
    )(aggP, hP, recT, exlT, b3, linW, linb)


def _mlp_tc(gsrc, gdst, alpha, a1, a2, a3, b1, w2, b2):
    eblk = 2000

    def body(g1_ref, g2_ref, al_ref, a1_ref, a2_ref, a3_ref, b1_ref,
             w2_ref, b2_ref, o_ref):
        hid = (jnp.dot(g1_ref[...], a1_ref[...], preferred_element_type=_f32)
               + jnp.dot(al_ref[...], a2_ref[...], preferred_element_type=_f32)
               + jnp.dot(g2_ref[...], a3_ref[...], preferred_element_type=_f32)
               + b1_ref[...])
        hid = jnp.maximum(hid, 0.0)
        o_ref[...] = jnp.dot(hid, w2_ref[...],
                             preferred_element_type=_f32) + b2_ref[...]

    return pl.pallas_call(
        body,
        grid=(E // eblk,),
        in_specs=[pl.BlockSpec((eblk, 64), lambda i: (i, 0)),
                  pl.BlockSpec((eblk, 64), lambda i: (i, 0)),
                  pl.BlockSpec((eblk, 8), lambda i: (i, 0)),
                  pl.BlockSpec((64, 256), lambda i: (0, 0)),
                  pl.BlockSpec((8, 256), lambda i: (0, 0)),
                  pl.BlockSpec((64, 256), lambda i: (0, 0)),
                  pl.BlockSpec((1, 256), lambda i: (0, 0)),
                  pl.BlockSpec((256, 2), lambda i: (0, 0)),
                  pl.BlockSpec((1, 2), lambda i: (0, 0))],
        out_specs=pl.BlockSpec((eblk, 2), lambda i: (i, 0)),
        out_shape=jax.ShapeDtypeStruct((E, 2), _f32),
    )(gsrc, gdst, alpha, a1, a2, a3, b1, w2, b2)


def kernel(x, edge_index, edge_attr, return_attention_weights, params):
    p = params
    src = edge_index[0]
    dst = edge_index[1]
    eattr = edge_attr[:, 0]
    xp = jnp.pad(x, ((0, NP - N), (0, 0)))
    zrow = jnp.zeros((64, 128), _f32)

    est = _estats_tc(eattr.reshape(ROWS, 128))
    emin, emax, emean = est[0, 0], est[1, 0], est[2, 0]

    z0, skipP = _prep_tc(xp, p["ne_W"], p["ne_b"], p["skip_W"], p["skip_b"])

    def layer(feats, pp, first):
        w = pp["W"]
        ind = w.shape[0]
        us = (w.reshape(ind, H, C) * pp["att_src"][None]).sum(-1)
        ud = (w.reshape(ind, H, C) * pp["att_dst"][None]).sum(-1)
        ucat = jnp.concatenate([us, ud], axis=1)
        ve = (pp["W_e"].reshape(64, H, C) * pp["att_e"][None]).sum(-1)
        mvec = (p["ee_W"] @ ve)[0]
        cvec = p["ee_b"] @ ve
        ael = emean * mvec + cvec

        if first:
            hP = _pre1_tc(feats, w)
            ascols = _asd1_tc(feats, ucat)
        else:
            hP = _pre_tc(feats, w)
            ascols = _asd_tc(feats, ucat)

        asrc = ascols[:N, :8]
        adst = ascols[:N, 8:]
        ae_max = jnp.maximum(jnp.where(mvec > 0, mvec * emax, mvec * emin)
                             + cvec, ael)
        b = jnp.max(asrc, axis=0) + jnp.max(adst, axis=0) + ae_max
        bl = jnp.where(b >= 0.0, b, 0.2 * b)

        psplat = jnp.stack([
            jnp.broadcast_to(mvec[:, None], (8, 16)),
            jnp.broadcast_to(cvec[:, None], (8, 16)),
            jnp.broadcast_to(bl[:, None], (8, 16))])

        asrcT = jnp.copy(ascols[:, :8].T)
        adstT = jnp.copy(ascols[:, 8:].T)
        exT, den2 = _kernel_a(src, dst, eattr, asrcT, adstT, psplat)

        den = (den2[0] + den2[1]).reshape(8, NP)
        denC = jnp.copy(den.T)
        consts = jnp.stack([ael, bl])
        recT, exlT = _dens_tc(ascols, denC, consts)

        hflat = hP.reshape(4 * NP, 128)
        aggP = _kernel_b(src, dst, exT, hflat, zrow)
        return hP, aggP, recT, exlT, exT

    hP, aggP, recT, exlT, _ = layer(z0, p["c1"], True)
    b1p = p["c1"]["b"].reshape(4, 128)
    oP = _post_tc(aggP, hP, recT, exlT, skipP, b1p)

    hP, aggP, recT, exlT, _ = layer(oP, p["c2"], False)
    b2p = p["c2"]["b"].reshape(4, 128)
    oP = _post_tc(aggP, hP, recT, exlT, skipP, b2p)

    hP, aggP, recT, exlT, exT3 = layer(oP, p["c3"], False)
    emb, xoutp = _post3_tc(aggP, hP, recT, exlT,
                           p["c3"]["b"].reshape(1, 64), p["lin_W"],
                           p["lin_b"].reshape(1, 2))

    rec3pl = jnp.copy(recT.T)
    gsrcf, gdstf, alphaT = _kernel_c(src, dst, emb, exT3, rec3pl)
    gsrc = gsrcf.reshape(E, 64)
    gdst = gdstf.reshape(E, 64)
    alpha = alphaT.T

    w1 = p["mlp_W1"]
    edge_out = _mlp_tc(gsrc, gdst, alpha,
                       w1[:64], w1[64:72], w1[72:],
                       p["mlp_b1"].reshape(1, 256), p["mlp_W2"],
                       p["mlp_b2"].reshape(1, 2))
    return xoutp[:N], edge_out


# speedup vs baseline: 23.9135x; 1.1323x over previous
import functools

import jax
import jax.numpy as jnp
from jax import lax
from jax.experimental import pallas as pl
from jax.experimental.pallas import tpu as pltpu
from jax.experimental.pallas import tpu_sc as plsc

N = 10000
E = 320000
H = 8
C = 64
NP = 10240
ROWS = E // 128
RSC = ROWS // 2
BLK = 1280
NBLK = NP // BLK

_f32 = jnp.float32
_i32 = jnp.int32


def _ka_body(src_hbm, dst_hbm, ea_hbm, asrcT_hbm, adstT_hbm, ps_hbm,
             exT_hbm, den_hbm,
             asrc_v, adst_v, srcb, dstb, eab, exs, idxs, pbuf, zbuf,
             sem_w, sem_s, den_sp):
    c = lax.axis_index("c")
    s = lax.axis_index("s")

    @pl.loop(0, 64)
    def _(i):
        zbuf[pl.ds(i * 16, 16)] = jnp.zeros((16,), _f32)

    base0 = s * 5120
    for i in range(5):
        pltpu.sync_copy(zbuf, den_sp.at[pl.ds(base0 + i * 1024, 1024)])
    pltpu.sync_copy(ps_hbm, pbuf)
    plsc.subcore_barrier()

    for g in range(2):
        pltpu.sync_copy(asrcT_hbm.at[pl.ds(4 * g * NP, 4 * NP)], asrc_v)
        pltpu.sync_copy(adstT_hbm.at[pl.ds(4 * g * NP, 4 * NP)], adst_v)
        ms = [pbuf[0, 4 * g + h4] for h4 in range(4)]
        cs = [pbuf[1, 4 * g + h4] for h4 in range(4)]
        bs = [pbuf[2, 4 * g + h4] for h4 in range(4)]

        def superchunk(r0, nr, g=g, ms=ms, cs=cs, bs=bs):
            base = r0 * 128
            ne = nr * 128
            pltpu.sync_copy(src_hbm.at[pl.ds(base, ne)],
                            srcb.at[pl.ds(0, ne)])
            pltpu.sync_copy(dst_hbm.at[pl.ds(base, ne)],
                            dstb.at[pl.ds(0, ne)])
            pltpu.sync_copy(ea_hbm.at[pl.ds(base, ne)],
                            eab.at[pl.ds(0, ne)])

            for r in range(nr):
                @pl.loop(0, 8)
                def _(j8, r=r):
                    j = r * 8 + j8
                    sv = srcb[pl.ds(j * 16, 16)]
                    dv = dstb[pl.ds(j * 16, 16)]
                    ev = eab[pl.ds(j * 16, 16)]
                    for h4 in range(4):
                        h = 4 * g + h4
                        av = plsc.load_gather(asrc_v, [sv + h4 * NP])
                        bv = plsc.load_gather(adst_v, [dv + h4 * NP])
                        lg = av + bv + ms[h4] * ev + cs[h4]
                        lg = jnp.maximum(lg, 0.2 * lg)
                        exv = jnp.exp(lg - bs[h4])
                        exs[h4, pl.ds(j * 16, 16)] = exv
                        idxs[h4, r, pl.ds(j8 * 16, 16)] = dv + h * NP

            for h4 in range(4):
                h = 4 * g + h4
                pltpu.async_copy(exs.at[h4, pl.ds(0, ne)],
                                 exT_hbm.at[h, pl.ds(base, ne)], sem_w)
                for r in range(nr):
                    pltpu.async_copy(exs.at[h4, pl.ds(r * 128, 128)],
                                     den_sp.at[idxs.at[h4, r]], sem_s,
                                     add=True)
                for r in range(nr):
                    pltpu.make_async_copy(exs.at[h4, pl.ds(r * 128, 128)],
                                          den_sp.at[idxs.at[h4, r]],
                                          sem_s).wait()
                pltpu.make_async_copy(exs.at[h4, pl.ds(0, ne)],
                                      exT_hbm.at[h, pl.ds(base, ne)],
                                      sem_w).wait()

        tile_row0 = c * RSC + s * 78

        @pl.loop(0, 9)
        def _(k):
            superchunk(tile_row0 + k * 8, 8)

        superchunk(tile_row0 + 72, 6)

        @pl.when(s < 2)
        def _():
            superchunk(c * RSC + 1248 + s, 1)

    plsc.subcore_barrier()
    pltpu.sync_copy(den_sp.at[pl.ds(s * 5120, 5120)],
                    den_hbm.at[c, pl.ds(s * 5120, 5120)])


def _kernel_a(src, dst, eattr, asrcT, adstT, psplat):
    return pl.kernel(
        _ka_body,
        out_type=[jax.ShapeDtypeStruct((H, E), _f32),
                  jax.ShapeDtypeStruct((2, 8 * NP), _f32)],
        mesh=plsc.VectorSubcoreMesh(core_axis_name="c", subcore_axis_name="s"),
        compiler_params=pltpu.CompilerParams(needs_layout_passes=False),
        scratch_types=[
            pltpu.VMEM((4 * NP,), _f32),
            pltpu.VMEM((4 * NP,), _f32),
            pltpu.VMEM((1024,), _i32),
            pltpu.VMEM((1024,), _i32),
            pltpu.VMEM((1024,), _f32),
            pltpu.VMEM((4, 1024), _f32),
            pltpu.VMEM((4, 8, 128), _i32),
            pltpu.VMEM((3, 8, 16), _f32),
            pltpu.VMEM((1024,), _f32),
            pltpu.SemaphoreType.DMA,
            pltpu.SemaphoreType.DMA,
            pltpu.VMEM_SHARED((8 * NP,), _f32),
        ],
    )(src, dst, eattr, asrcT.reshape(8 * NP), adstT.reshape(8 * NP), psplat)


def _kb_body(src_hbm, dst_hbm, exT_hbm, hflat_hbm, zrow_hbm,
             agg_hbm,
             srcb, dstb, exa, exb, srcp2, dstb2,
             g0, g1, sg0, sg1, ss0, ss1, acc_sp):
    c = lax.axis_index("c")
    s = lax.axis_index("s")
    gbufs = (g0, g1)
    sgs = (sg0, sg1)
    sss = (ss0, ss1)
    NR = 2

    for q in range(2):
        p = 2 * c + q
        for i in range(10):
            pltpu.sync_copy(zrow_hbm, acc_sp.at[pl.ds(s * 640 + i * 64, 64)])
        plsc.subcore_barrier()
        off = p * NP
        row0 = s * 156

        def scale(gb, exav, exbv, jdx0):
            for jj in range(16):
                jdx = jdx0 + jj
                sa = exav[jj]
                sb = exbv[jj]
                for cc in range(4):
                    gb[jdx, pl.ds(cc * 16, 16)] = (
                        gb[jdx, pl.ds(cc * 16, 16)] * sa)
                for cc in range(4, 8):
                    gb[jdx, pl.ds(cc * 16, 16)] = (
                        gb[jdx, pl.ds(cc * 16, 16)] * sb)

        @pl.loop(0, 78)
        def _(sk):
            r0 = row0 + sk * NR
            base = r0 * 128
            pltpu.sync_copy(src_hbm.at[pl.ds(base, 256)], srcb)
            pltpu.sync_copy(dst_hbm.at[pl.ds(base, 256)], dstb)
            pltpu.sync_copy(exT_hbm.at[2 * p, pl.ds(base, 256)], exa)
            pltpu.sync_copy(exT_hbm.at[2 * p + 1, pl.ds(base, 256)], exb)

            @pl.when(sk > 0)
            def _():
                for r in range(NR):
                    pltpu.make_async_copy(
                        gbufs[r], acc_sp.at[dstb2.at[r]], sss[r]).wait()

            for r in range(NR):
                for j in range(8):
                    srcp2[r, pl.ds(j * 16, 16)] = (
                        srcb[pl.ds(r * 128 + j * 16, 16)] + off)
                    dstb2[r, pl.ds(j * 16, 16)] = (
                        dstb[pl.ds(r * 128 + j * 16, 16)])
            for r in range(NR):
                pltpu.async_copy(hflat_hbm.at[srcp2.at[r]], gbufs[r], sgs[r])
            for r in range(NR):
                pltpu.make_async_copy(
                    hflat_hbm.at[srcp2.at[r]], gbufs[r], sgs[r]).wait()

                @pl.loop(0, 8)
                def _(j16, r=r):
                    exav = exa[pl.ds(r * 128 + j16 * 16, 16)]
                    exbv = exb[pl.ds(r * 128 + j16 * 16, 16)]
                    scale(gbufs[r], exav, exbv, j16 * 16)

                pltpu.async_copy(gbufs[r], acc_sp.at[dstb2.at[r]], sss[r],
                                 add=True)

        for r in range(NR):
            pltpu.make_async_copy(
                gbufs[r], acc_sp.at[dstb2.at[r]], sss[r]).wait()

        @pl.when(s < 4)
        def _():
            base = (2496 + s) * 128
            pltpu.sync_copy(src_hbm.at[pl.ds(base, 128)],
                            srcb.at[pl.ds(0, 128)])
            pltpu.sync_copy(dst_hbm.at[pl.ds(base, 128)],
                            dstb.at[pl.ds(0, 128)])
            pltpu.sync_copy(exT_hbm.at[2 * p, pl.ds(base, 128)],
                            exa.at[pl.ds(0, 128)])
            pltpu.sync_copy(exT_hbm.at[2 * p + 1, pl.ds(base, 128)],
                            exb.at[pl.ds(0, 128)])
            for j in range(8):
                srcp2[0, pl.ds(j * 16, 16)] = srcb[pl.ds(j * 16, 16)] + off
                dstb2[0, pl.ds(j * 16, 16)] = dstb[pl.ds(j * 16, 16)]
            pltpu.sync_copy(hflat_hbm.at[srcp2.at[0]], g0)

            @pl.loop(0, 8)
            def _(j16):
                exav = exa[pl.ds(j16 * 16, 16)]
                exbv = exb[pl.ds(j16 * 16, 16)]
                scale(g0, exav, exbv, j16 * 16)

            pltpu.sync_copy(g0, acc_sp.at[dstb2.at[0]], add=True)

        plsc.subcore_barrier()
        pltpu.sync_copy(acc_sp.at[pl.ds(s * 640, 640)],
                        agg_hbm.at[p, pl.ds(s * 640, 640)])
        plsc.subcore_barrier()


def _kernel_b(src, dst, exT, hflat, zrow):
    return pl.kernel(
        _kb_body,
        out_type=[jax.ShapeDtypeStruct((4, NP, 128), _f32)],
        mesh=plsc.VectorSubcoreMesh(core_axis_name="c", subcore_axis_name="s"),
        compiler_params=pltpu.CompilerParams(needs_layout_passes=False),
        scratch_types=(
            [pltpu.VMEM((256,), _i32),
             pltpu.VMEM((256,), _i32),
             pltpu.VMEM((256,), _f32),
             pltpu.VMEM((256,), _f32),
             pltpu.VMEM((2, 128), _i32),
             pltpu.VMEM((2, 128), _i32)]
            + [pltpu.VMEM((128, 128), _f32) for _ in range(2)]
            + [pltpu.SemaphoreType.DMA for _ in range(4)]
            + [pltpu.VMEM_SHARED((NP, 128), _f32)]),
    )(src, dst, exT, hflat, zrow)[0]


def _kc_body(src_hbm, dst_hbm, emb_hbm, exT_hbm, rec_hbm,
             gsrc_hbm, gdst_hbm, alphaT_hbm,
             srcb, dstb, gsa, gsd, packb, exc, alpb, sga, sgd, sem_w, recv):
    c = lax.axis_index("c")
    s = lax.axis_index("s")
    pltpu.sync_copy(rec_hbm, recv)

    def chunk(row):
        base = row * 128
        pltpu.sync_copy(src_hbm.at[pl.ds(base, 128)], srcb)
        pltpu.sync_copy(dst_hbm.at[pl.ds(base, 128)], dstb)

        def gather_pack(idxb, gb, out_hbm):
            pltpu.sync_copy(emb_hbm.at[idxb], gb)

            @pl.loop(0, 8)
            def _(j16):
                for jj in range(16):
                    j = j16 * 16 + jj
                    for cc in range(4):
                        packb[pl.ds(j * 64 + cc * 16, 16)] = (
                            gb[j, pl.ds(cc * 16, 16)])

            pltpu.sync_copy(packb, out_hbm.at[pl.ds(base * 64, 8192)])

        gather_pack(srcb, gsa, gsrc_hbm)
        gather_pack(dstb, gsd, gdst_hbm)

        for h in range(8):
            pltpu.sync_copy(exT_hbm.at[h, pl.ds(base, 128)], exc.at[h])

        @pl.loop(0, 8)
        def _(j):
            dv = dstb[pl.ds(j * 16, 16)]
            for h in range(8):
                rv = plsc.load_gather(recv, [dv + h * NP])
                ev = exc[h, pl.ds(j * 16, 16)]
                alpb[h, pl.ds(j * 16, 16)] = ev * rv

        for h in range(8):
            pltpu.sync_copy(alpb.at[h], alphaT_hbm.at[h, pl.ds(base, 128)])

    row0 = c * RSC + s * 78

    @pl.loop(0, 78)
    def _(k):
        chunk(row0 + k)

    @pl.when(s < 2)
    def _():
        chunk(c * RSC + 1248 + s)


def _kernel_c(src, dst, emb2, exT, recpl):
    return pl.kernel(
        _kc_body,
        out_type=[jax.ShapeDtypeStruct((E * 64,), _f32),
                  jax.ShapeDtypeStruct((E * 64,), _f32),
                  jax.ShapeDtypeStruct((H, E), _f32)],
        mesh=plsc.VectorSubcoreMesh(core_axis_name="c", subcore_axis_name="s"),
        compiler_params=pltpu.CompilerParams(needs_layout_passes=False),
        scratch_types=[
            pltpu.VMEM((128,), _i32),
            pltpu.VMEM((128,), _i32),
            pltpu.VMEM((128, 128), _f32),
            pltpu.VMEM((128, 128), _f32),
            pltpu.VMEM((8192,), _f32),
            pltpu.VMEM((8, 128), _f32),
            pltpu.VMEM((8, 128), _f32),
            pltpu.SemaphoreType.DMA,
            pltpu.SemaphoreType.DMA,
            pltpu.SemaphoreType.DMA,
            pltpu.VMEM((8 * NP,), _f32),
        ],
    )(src, dst, emb2, exT, recpl.reshape(8 * NP))


def _prep_tc(xp, neW, neb, skW, skb):
    def body(x_ref, nw_ref, nb_ref, sw_ref, sb_ref, z0_ref, skp_ref):
        xb = x_ref[...]
        p = pl.program_id(0)
        z0_ref[...] = jnp.dot(xb, nw_ref[...],
                              preferred_element_type=_f32) + nb_ref[...]
        skp_ref[0] = jnp.dot(xb, sw_ref[...],
                             preferred_element_type=_f32) + sb_ref[pl.ds(p, 1)]

    return pl.pallas_call(
        body,
        grid=(4, NBLK),
        in_specs=[pl.BlockSpec((BLK, 3), lambda p, i: (i, 0)),
                  pl.BlockSpec((3, 64), lambda p, i: (0, 0)),
                  pl.BlockSpec((1, 64), lambda p, i: (0, 0)),
                  pl.BlockSpec((3, 128), lambda p, i: (0, p)),
                  pl.BlockSpec((4, 128), lambda p, i: (0, 0))],
        out_specs=[pl.BlockSpec((BLK, 64), lambda p, i: (i, 0)),
                   pl.BlockSpec((1, BLK, 128), lambda p, i: (p, i, 0))],
        out_shape=[jax.ShapeDtypeStruct((NP, 64), _f32),
                   jax.ShapeDtypeStruct((4, NP, 128), _f32)],
    )(xp, neW, neb.reshape(1, 64), skW, skb.reshape(4, 128))


def _estats_tc(e2d):
    def body(e_ref, o_ref):
        v = e_ref[...]
        o_ref[0, :] = jnp.full((128,), jnp.min(v), _f32)
        o_ref[1, :] = jnp.full((128,), jnp.max(v), _f32)
        o_ref[2, :] = jnp.full((128,), jnp.mean(v), _f32)
        o_ref[3, :] = jnp.zeros((128,), _f32)

    return pl.pallas_call(
        body,
        out_shape=jax.ShapeDtypeStruct((4, 128), _f32),
    )(e2d)


def _asd1_tc(z0, ucat):
    def body(z_ref, u_ref, o_ref):
        o_ref[...] = jnp.dot(z_ref[...], u_ref[...],
                             preferred_element_type=_f32)

    return pl.pallas_call(
        body,
        grid=(NBLK,),
        in_specs=[pl.BlockSpec((BLK, 64), lambda i: (i, 0)),
                  pl.BlockSpec((64, 16), lambda i: (0, 0))],
        out_specs=pl.BlockSpec((BLK, 16), lambda i: (i, 0)),
        out_shape=jax.ShapeDtypeStruct((NP, 16), _f32),
    )(z0, ucat)


def _asd_tc(oP, ucat):
    def body(o_ref, u_ref, out_ref):
        part = jnp.dot(o_ref[0], u_ref[...], preferred_element_type=_f32)

        @pl.when(pl.program_id(1) == 0)
        def _():
            out_ref[...] = jnp.zeros_like(out_ref)

        out_ref[...] += part

    return pl.pallas_call(
        body,
        grid=(NBLK, 4),
        in_specs=[pl.BlockSpec((1, BLK, 128), lambda i, p: (p, i, 0)),
                  pl.BlockSpec((128, 16), lambda i, p: (p, 0))],
        out_specs=pl.BlockSpec((BLK, 16), lambda i, p: (i, 0)),
        out_shape=jax.ShapeDtypeStruct((NP, 16), _f32),
    )(oP, ucat)


def _pre1_tc(z0, w):
    def body(z_ref, w_ref, h_ref):
        h_ref[0] = jnp.dot(z_ref[...], w_ref[...],
                           preferred_element_type=_f32)

    return pl.pallas_call(
        body,
        grid=(4, NBLK),
        in_specs=[pl.BlockSpec((BLK, 64), lambda q, i: (i, 0)),
                  pl.BlockSpec((64, 128), lambda q, i: (0, q))],
        out_specs=pl.BlockSpec((1, BLK, 128), lambda q, i: (q, i, 0)),
        out_shape=jax.ShapeDtypeStruct((4, NP, 128), _f32),
    )(z0, w)


def _pre_tc(oP, w):
    def body(o_ref, w_ref, h_ref):
        part = jnp.dot(o_ref[0], w_ref[...], preferred_element_type=_f32)

        @pl.when(pl.program_id(2) == 0)
        def _():
            h_ref[...] = jnp.zeros_like(h_ref)

        h_ref[0] += part

    return pl.pallas_call(
        body,
        grid=(4, NBLK, 4),
        in_specs=[pl.BlockSpec((1, BLK, 128), lambda q, i, p: (p, i, 0)),
                  pl.BlockSpec((128, 128), lambda q, i, p: (p, q))],
        out_specs=pl.BlockSpec((1, BLK, 128), lambda q, i, p: (q, i, 0)),
        out_shape=jax.ShapeDtypeStruct((4, NP, 128), _f32),
    )(oP, w)


def _dens_tc(ascols, denC, consts):
    def body(a_ref, d_ref, c_ref, rec_ref, exl_ref):
        a = a_ref[...]
        l = a[:, :8] + a[:, 8:] + c_ref[0:1, :]
        lr = jnp.maximum(l, 0.2 * l)
        exl = jnp.exp(lr - c_ref[1:2, :])
        exl_ref[...] = exl
        rec_ref[...] = 1.0 / (d_ref[...] + exl + 1e-16)

    return pl.pallas_call(
        body,
        grid=(NBLK,),
        in_specs=[pl.BlockSpec((BLK, 16), lambda i: (i, 0)),
                  pl.BlockSpec((BLK, 8), lambda i: (i, 0)),
                  pl.BlockSpec((2, 8), lambda i: (0, 0))],
        out_specs=[pl.BlockSpec((BLK, 8), lambda i: (i, 0)),
                   pl.BlockSpec((BLK, 8), lambda i: (i, 0))],
        out_shape=[jax.ShapeDtypeStruct((NP, 8), _f32),
                   jax.ShapeDtypeStruct((NP, 8), _f32)],
    )(ascols, denC, consts)


def _post_tc(aggP, hP, recT, exlT, skipP, bP):
    def body(g_ref, h_ref, r_ref, x_ref, s_ref, b_ref, o_ref):
        for p in range(4):
            rec2 = r_ref[:, 2 * p:2 * p + 2]
            exl2 = x_ref[:, 2 * p:2 * p + 2]
            recs = jnp.concatenate(
                [jnp.broadcast_to(rec2[:, 0:1], (BLK, 64)),
                 jnp.broadcast_to(rec2[:, 1:2], (BLK, 64))], axis=1)
            exls = jnp.concatenate(
                [jnp.broadcast_to(exl2[:, 0:1], (BLK, 64)),
                 jnp.broadcast_to(exl2[:, 1:2], (BLK, 64))], axis=1)
            v = (g_ref[p] + h_ref[p] * exls) * recs + b_ref[p:p + 1]
            o_ref[p] = jnp.where(v > 0, v, (jnp.exp(v) - 1.0)) + s_ref[p]

    return pl.pallas_call(
        body,
        grid=(NBLK,),
        in_specs=[pl.BlockSpec((4, BLK, 128), lambda i: (0, i, 0)),
                  pl.BlockSpec((4, BLK, 128), lambda i: (0, i, 0)),
                  pl.BlockSpec((BLK, 8), lambda i: (i, 0)),
                  pl.BlockSpec((BLK, 8), lambda i: (i, 0)),
                  pl.BlockSpec((4, BLK, 128), lambda i: (0, i, 0)),
                  pl.BlockSpec((4, 128), lambda i: (0, 0))],
        out_specs=pl.BlockSpec((4, BLK, 128), lambda i: (0, i, 0)),
        out_shape=jax.ShapeDtypeStruct((4, NP, 128), _f32),
    )(aggP, hP, recT, exlT, skipP, bP)


def _post3_tc(aggP, hP, recT, exlT, b3, linW, linb):
    def body(g_ref, h_ref, r_ref, x_ref, b_ref, lw_ref, lb_ref,
             emb_ref, xo_ref):
        ssum = jnp.zeros((BLK, 64), _f32)
        for p in range(4):
            for half in range(2):
                hh = 2 * p + half
                seg = g_ref[p, :, 64 * half:64 * half + 64]
                hseg = h_ref[p, :, 64 * half:64 * half + 64]
                ssum += (seg + hseg * x_ref[:, hh:hh + 1]) * r_ref[:, hh:hh + 1]
        o = ssum * 0.125 + b_ref[...]
        embv = jnp.where(o > 0, o, (jnp.exp(o) - 1.0))
        emb_ref[...] = jnp.concatenate(
            [embv, jnp.zeros((BLK, 64), _f32)], axis=1)
        xo_ref[...] = jnp.dot(embv, lw_ref[...],
                              preferred_element_type=_f32) + lb_ref[...]

    return pl.pallas_call(
        body,
        grid=(NBLK,),
        in_specs=[pl.BlockSpec((4, BLK, 128), lambda i: (0, i, 0)),
                  pl.BlockSpec((4, BLK, 128), lambda i: (0, i, 0)),
                  pl.BlockSpec((BLK, 8), lambda i: (i, 0)),
                  pl.BlockSpec((BLK, 8), lambda i: (i, 0)),
                  pl.BlockSpec((1, 64), lambda i: (0, 0)),
                  pl.BlockSpec((64, 2), lambda i: (0, 0)),
                  pl.BlockSpec((1, 2), lambda i: (0, 0))],
        out_specs=[pl.BlockSpec((BLK, 128), lambda i: (i, 0)),
                   pl.BlockSpec((BLK, 2), lambda i: (i, 0))],
        out_shape=[jax.ShapeDtypeStruct((NP, 128), _f32),
                   jax.ShapeDtypeStruct((NP, 2), _f32)],
    )(aggP, hP, recT, exlT, b3, linW, linb)


def _mlp_tc(gsrc, gdst, alpha, a1, a2, a3, b1, w2, b2):
    eblk = 2000

    def body(g1_ref, g2_ref, al_ref, a1_ref, a2_ref, a3_ref, b1_ref,
             w2_ref, b2_ref, o_ref):
        hid = (jnp.dot(g1_ref[...], a1_ref[...], preferred_element_type=_f32)
               + jnp.dot(al_ref[...], a2_ref[...], preferred_element_type=_f32)
               + jnp.dot(g2_ref[...], a3_ref[...], preferred_element_type=_f32)
               + b1_ref[...])
        hid = jnp.maximum(hid, 0.0)
        o_ref[...] = jnp.dot(hid, w2_ref[...],
                             preferred_element_type=_f32) + b2_ref[...]

    return pl.pallas_call(
        body,
        grid=(E // eblk,),
        in_specs=[pl.BlockSpec((eblk, 64), lambda i: (i, 0)),
                  pl.BlockSpec((eblk, 64), lambda i: (i, 0)),
                  pl.BlockSpec((eblk, 8), lambda i: (i, 0)),
                  pl.BlockSpec((64, 256), lambda i: (0, 0)),
                  pl.BlockSpec((8, 256), lambda i: (0, 0)),
                  pl.BlockSpec((64, 256), lambda i: (0, 0)),
                  pl.BlockSpec((1, 256), lambda i: (0, 0)),
                  pl.BlockSpec((256, 2), lambda i: (0, 0)),
                  pl.BlockSpec((1, 2), lambda i: (0, 0))],
        out_specs=pl.BlockSpec((eblk, 2), lambda i: (i, 0)),
        out_shape=jax.ShapeDtypeStruct((E, 2), _f32),
    )(gsrc, gdst, alpha, a1, a2, a3, b1, w2, b2)


def kernel(x, edge_index, edge_attr, return_attention_weights, params):
    p = params
    src = edge_index[0]
    dst = edge_index[1]
    eattr = edge_attr[:, 0]
    xp = jnp.pad(x, ((0, NP - N), (0, 0)))
    zrow = jnp.zeros((64, 128), _f32)

    est = _estats_tc(eattr.reshape(ROWS, 128))
    emin, emax, emean = est[0, 0], est[1, 0], est[2, 0]

    z0, skipP = _prep_tc(xp, p["ne_W"], p["ne_b"], p["skip_W"], p["skip_b"])

    def layer(feats, pp, first):
        w = pp["W"]
        ind = w.shape[0]
        us = (w.reshape(ind, H, C) * pp["att_src"][None]).sum(-1)
        ud = (w.reshape(ind, H, C) * pp["att_dst"][None]).sum(-1)
        ucat = jnp.concatenate([us, ud], axis=1)
        ve = (pp["W_e"].reshape(64, H, C) * pp["att_e"][None]).sum(-1)
        mvec = (p["ee_W"] @ ve)[0]
        cvec = p["ee_b"] @ ve
        ael = emean * mvec + cvec

        if first:
            hP = _pre1_tc(feats, w)
            ascols = _asd1_tc(feats, ucat)
        else:
            hP = _pre_tc(feats, w)
            ascols = _asd_tc(feats, ucat)

        asrc = ascols[:N, :8]
        adst = ascols[:N, 8:]
        ae_max = jnp.maximum(jnp.where(mvec > 0, mvec * emax, mvec * emin)
                             + cvec, ael)
        b = jnp.max(asrc, axis=0) + jnp.max(adst, axis=0) + ae_max
        bl = jnp.where(b >= 0.0, b, 0.2 * b)

        psplat = jnp.stack([
            jnp.broadcast_to(mvec[:, None], (8, 16)),
            jnp.broadcast_to(cvec[:, None], (8, 16)),
            jnp.broadcast_to(bl[:, None], (8, 16))])

        asrcT = jnp.copy(ascols[:, :8].T)
        adstT = jnp.copy(ascols[:, 8:].T)
        exT, den2 = _kernel_a(src, dst, eattr, asrcT, adstT, psplat)

        den = (den2[0] + den2[1]).reshape(8, NP)
        denC = jnp.copy(den.T)
        consts = jnp.stack([ael, bl])
        recT, exlT = _dens_tc(ascols, denC, consts)

        hflat = hP.reshape(4 * NP, 128)
        aggP = _kernel_b(src, dst, exT, hflat, zrow)
        return hP, aggP, recT, exlT, exT

    hP, aggP, recT, exlT, _ = layer(z0, p["c1"], True)
    b1p = p["c1"]["b"].reshape(4, 128)
    oP = _post_tc(aggP, hP, recT, exlT, skipP, b1p)

    hP, aggP, recT, exlT, _ = layer(oP, p["c2"], False)
    b2p = p["c2"]["b"].reshape(4, 128)
    oP = _post_tc(aggP, hP, recT, exlT, skipP, b2p)

    hP, aggP, recT, exlT, exT3 = layer(oP, p["c3"], False)
    emb, xoutp = _post3_tc(aggP, hP, recT, exlT,
                           p["c3"]["b"].reshape(1, 64), p["lin_W"],
                           p["lin_b"].reshape(1, 2))

    rec3pl = jnp.copy(recT.T)
    gsrcf, gdstf, alphaT = _kernel_c(src, dst, emb, exT3, rec3pl)
    gsrc = gsrcf.reshape(E, 64)
    gdst = gdstf.reshape(E, 64)
    alpha = alphaT.T

    w1 = p["mlp_W1"]
    edge_out = _mlp_tc(gsrc, gdst, alpha,
                       w1[:64], w1[64:72], w1[72:],
                       p["mlp_b1"].reshape(1, 256), p["mlp_W2"],
                       p["mlp_b2"].reshape(1, 2))
    return xoutp[:N], edge_out


# speedup vs baseline: 24.8267x; 1.0382x over previous
import functools

import jax
import jax.numpy as jnp
from jax import lax
from jax.experimental import pallas as pl
from jax.experimental.pallas import tpu as pltpu
from jax.experimental.pallas import tpu_sc as plsc

N = 10000
E = 320000
H = 8
C = 64
NP = 10240
ROWS = E // 128
RSC = ROWS // 2
BLK = 1280
NBLK = NP // BLK

_f32 = jnp.float32
_i32 = jnp.int32


def _ka_body(src_hbm, dst_hbm, ea_hbm, asrcT_hbm, adstT_hbm, ps_hbm,
             exT_hbm, den_hbm,
             asrc_v, adst_v, srcb, dstb, eab, exs, idxs, pbuf, zbuf,
             sem_w, sem_s, den_sp):
    c = lax.axis_index("c")
    s = lax.axis_index("s")

    @pl.loop(0, 64)
    def _(i):
        zbuf[pl.ds(i * 16, 16)] = jnp.zeros((16,), _f32)

    base0 = s * 5120
    for i in range(5):
        pltpu.sync_copy(zbuf, den_sp.at[pl.ds(base0 + i * 1024, 1024)])
    pltpu.sync_copy(ps_hbm, pbuf)
    plsc.subcore_barrier()

    for g in range(2):
        pltpu.sync_copy(asrcT_hbm.at[pl.ds(4 * g * NP, 4 * NP)], asrc_v)
        pltpu.sync_copy(adstT_hbm.at[pl.ds(4 * g * NP, 4 * NP)], adst_v)
        ms = [pbuf[0, 4 * g + h4] for h4 in range(4)]
        cs = [pbuf[1, 4 * g + h4] for h4 in range(4)]
        bs = [pbuf[2, 4 * g + h4] for h4 in range(4)]

        def superchunk(r0, nr, g=g, ms=ms, cs=cs, bs=bs):
            base = r0 * 128
            ne = nr * 128
            pltpu.sync_copy(src_hbm.at[pl.ds(base, ne)],
                            srcb.at[pl.ds(0, ne)])
            pltpu.sync_copy(dst_hbm.at[pl.ds(base, ne)],
                            dstb.at[pl.ds(0, ne)])
            pltpu.sync_copy(ea_hbm.at[pl.ds(base, ne)],
                            eab.at[pl.ds(0, ne)])

            for r in range(nr):
                @pl.loop(0, 8)
                def _(j8, r=r):
                    j = r * 8 + j8
                    sv = srcb[pl.ds(j * 16, 16)]
                    dv = dstb[pl.ds(j * 16, 16)]
                    ev = eab[pl.ds(j * 16, 16)]
                    for h4 in range(4):
                        h = 4 * g + h4
                        av = plsc.load_gather(asrc_v, [sv + h4 * NP])
                        bv = plsc.load_gather(adst_v, [dv + h4 * NP])
                        lg = av + bv + ms[h4] * ev + cs[h4]
                        lg = jnp.maximum(lg, 0.2 * lg)
                        exv = jnp.exp(lg - bs[h4])
                        exs[h4, pl.ds(j * 16, 16)] = exv
                        idxs[h4, r, pl.ds(j8 * 16, 16)] = dv + h * NP

            for h4 in range(4):
                h = 4 * g + h4
                pltpu.async_copy(exs.at[h4, pl.ds(0, ne)],
                                 exT_hbm.at[h, pl.ds(base, ne)], sem_w)
                for r in range(nr):
                    pltpu.async_copy(exs.at[h4, pl.ds(r * 128, 128)],
                                     den_sp.at[idxs.at[h4, r]], sem_s,
                                     add=True)
                for r in range(nr):
                    pltpu.make_async_copy(exs.at[h4, pl.ds(r * 128, 128)],
                                          den_sp.at[idxs.at[h4, r]],
                                          sem_s).wait()
                pltpu.make_async_copy(exs.at[h4, pl.ds(0, ne)],
                                      exT_hbm.at[h, pl.ds(base, ne)],
                                      sem_w).wait()

        tile_row0 = c * RSC + s * 78

        @pl.loop(0, 9)
        def _(k):
            superchunk(tile_row0 + k * 8, 8)

        superchunk(tile_row0 + 72, 6)

        @pl.when(s < 2)
        def _():
            superchunk(c * RSC + 1248 + s, 1)

    plsc.subcore_barrier()
    pltpu.sync_copy(den_sp.at[pl.ds(s * 5120, 5120)],
                    den_hbm.at[c, pl.ds(s * 5120, 5120)])


def _kernel_a(src, dst, eattr, asrcT, adstT, psplat):
    return pl.kernel(
        _ka_body,
        out_type=[jax.ShapeDtypeStruct((H, E), _f32),
                  jax.ShapeDtypeStruct((2, 8 * NP), _f32)],
        mesh=plsc.VectorSubcoreMesh(core_axis_name="c", subcore_axis_name="s"),
        compiler_params=pltpu.CompilerParams(needs_layout_passes=False),
        scratch_types=[
            pltpu.VMEM((4 * NP,), _f32),
            pltpu.VMEM((4 * NP,), _f32),
            pltpu.VMEM((1024,), _i32),
            pltpu.VMEM((1024,), _i32),
            pltpu.VMEM((1024,), _f32),
            pltpu.VMEM((4, 1024), _f32),
            pltpu.VMEM((4, 8, 128), _i32),
            pltpu.VMEM((3, 8, 16), _f32),
            pltpu.VMEM((1024,), _f32),
            pltpu.SemaphoreType.DMA,
            pltpu.SemaphoreType.DMA,
            pltpu.VMEM_SHARED((8 * NP,), _f32),
        ],
    )(src, dst, eattr, asrcT.reshape(8 * NP), adstT.reshape(8 * NP), psplat)


def _kb_body(src_hbm, dst_hbm, exT_hbm, hflat_hbm, zrow_hbm,
             agg_hbm,
             srcb, dstb, exa, exb, srcp2, dstb2,
             g0, g1, sg0, sg1, ss0, ss1, acc_sp):
    c = lax.axis_index("c")
    s = lax.axis_index("s")
    gbufs = (g0, g1)
    sgs = (sg0, sg1)
    sss = (ss0, ss1)
    NR = 2

    for q in range(2):
        p = 2 * c + q
        for i in range(10):
            pltpu.sync_copy(zrow_hbm, acc_sp.at[pl.ds(s * 640 + i * 64, 64)])
        plsc.subcore_barrier()
        off = p * NP
        row0 = s * 156

        def scale(gb, exav, exbv, jdx0):
            for jj in range(16):
                jdx = jdx0 + jj
                sa = exav[jj]
                sb = exbv[jj]
                for cc in range(4):
                    gb[jdx, pl.ds(cc * 16, 16)] = (
                        gb[jdx, pl.ds(cc * 16, 16)] * sa)
                for cc in range(4, 8):
                    gb[jdx, pl.ds(cc * 16, 16)] = (
                        gb[jdx, pl.ds(cc * 16, 16)] * sb)

        @pl.loop(0, 78)
        def _(sk):
            r0 = row0 + sk * NR
            base = r0 * 128
            pltpu.sync_copy(src_hbm.at[pl.ds(base, 256)], srcb)
            pltpu.sync_copy(dst_hbm.at[pl.ds(base, 256)], dstb)
            pltpu.sync_copy(exT_hbm.at[2 * p, pl.ds(base, 256)], exa)
            pltpu.sync_copy(exT_hbm.at[2 * p + 1, pl.ds(base, 256)], exb)

            @pl.when(sk > 0)
            def _():
                for r in range(NR):
                    pltpu.make_async_copy(
                        gbufs[r], acc_sp.at[dstb2.at[r]], sss[r]).wait()

            for r in range(NR):
                for j in range(8):
                    srcp2[r, pl.ds(j * 16, 16)] = (
                        srcb[pl.ds(r * 128 + j * 16, 16)] + off)
                    dstb2[r, pl.ds(j * 16, 16)] = (
                        dstb[pl.ds(r * 128 + j * 16, 16)])
            for r in range(NR):
                pltpu.async_copy(hflat_hbm.at[srcp2.at[r]], gbufs[r], sgs[r])
            for r in range(NR):
                pltpu.make_async_copy(
                    hflat_hbm.at[srcp2.at[r]], gbufs[r], sgs[r]).wait()

                @pl.loop(0, 8)
                def _(j16, r=r):
                    exav = exa[pl.ds(r * 128 + j16 * 16, 16)]
                    exbv = exb[pl.ds(r * 128 + j16 * 16, 16)]
                    scale(gbufs[r], exav, exbv, j16 * 16)

                pltpu.async_copy(gbufs[r], acc_sp.at[dstb2.at[r]], sss[r],
                                 add=True)

        for r in range(NR):
            pltpu.make_async_copy(
                gbufs[r], acc_sp.at[dstb2.at[r]], sss[r]).wait()

        @pl.when(s < 4)
        def _():
            base = (2496 + s) * 128
            pltpu.sync_copy(src_hbm.at[pl.ds(base, 128)],
                            srcb.at[pl.ds(0, 128)])
            pltpu.sync_copy(dst_hbm.at[pl.ds(base, 128)],
                            dstb.at[pl.ds(0, 128)])
            pltpu.sync_copy(exT_hbm.at[2 * p, pl.ds(base, 128)],
                            exa.at[pl.ds(0, 128)])
            pltpu.sync_copy(exT_hbm.at[2 * p + 1, pl.ds(base, 128)],
                            exb.at[pl.ds(0, 128)])
            for j in range(8):
                srcp2[0, pl.ds(j * 16, 16)] = srcb[pl.ds(j * 16, 16)] + off
                dstb2[0, pl.ds(j * 16, 16)] = dstb[pl.ds(j * 16, 16)]
            pltpu.sync_copy(hflat_hbm.at[srcp2.at[0]], g0)

            @pl.loop(0, 8)
            def _(j16):
                exav = exa[pl.ds(j16 * 16, 16)]
                exbv = exb[pl.ds(j16 * 16, 16)]
                scale(g0, exav, exbv, j16 * 16)

            pltpu.sync_copy(g0, acc_sp.at[dstb2.at[0]], add=True)

        plsc.subcore_barrier()
        pltpu.sync_copy(acc_sp.at[pl.ds(s * 640, 640)],
                        agg_hbm.at[p, pl.ds(s * 640, 640)])
        plsc.subcore_barrier()


def _kernel_b(src, dst, exT, hflat, zrow):
    return pl.kernel(
        _kb_body,
        out_type=[jax.ShapeDtypeStruct((4, NP, 128), _f32)],
        mesh=plsc.VectorSubcoreMesh(core_axis_name="c", subcore_axis_name="s"),
        compiler_params=pltpu.CompilerParams(needs_layout_passes=False),
        scratch_types=(
            [pltpu.VMEM((256,), _i32),
             pltpu.VMEM((256,), _i32),
             pltpu.VMEM((256,), _f32),
             pltpu.VMEM((256,), _f32),
             pltpu.VMEM((2, 128), _i32),
             pltpu.VMEM((2, 128), _i32)]
            + [pltpu.VMEM((128, 128), _f32) for _ in range(2)]
            + [pltpu.SemaphoreType.DMA for _ in range(4)]
            + [pltpu.VMEM_SHARED((NP, 128), _f32)]),
    )(src, dst, exT, hflat, zrow)[0]


def _kc_body(src_hbm, dst_hbm, emb_hbm, exT_hbm, rec_hbm,
             gsrc_hbm, gdst_hbm, alphaT_hbm,
             srcb, dstb, gsa, gsd, packb, exc, alpb, sga, sgd, sem_w, recv):
    c = lax.axis_index("c")
    s = lax.axis_index("s")
    pltpu.sync_copy(rec_hbm, recv)

    def chunk(row):
        base = row * 128
        pltpu.sync_copy(src_hbm.at[pl.ds(base, 128)], srcb)
        pltpu.sync_copy(dst_hbm.at[pl.ds(base, 128)], dstb)
        pltpu.async_copy(emb_hbm.at[srcb], gsa, sga)
        pltpu.async_copy(emb_hbm.at[dstb], gsd, sgd)

        for h in range(8):
            pltpu.sync_copy(exT_hbm.at[h, pl.ds(base, 128)], exc.at[h])

        @pl.loop(0, 8)
        def _(j):
            dv = dstb[pl.ds(j * 16, 16)]
            for h in range(8):
                rv = plsc.load_gather(recv, [dv + h * NP])
                ev = exc[h, pl.ds(j * 16, 16)]
                alpb[h, pl.ds(j * 16, 16)] = ev * rv

        for h in range(8):
            pltpu.async_copy(alpb.at[h], alphaT_hbm.at[h, pl.ds(base, 128)],
                             sem_w)

        def gather_pack(idxb, gb, sem, out_hbm):
            pltpu.make_async_copy(emb_hbm.at[idxb], gb, sem).wait()

            @pl.loop(0, 8)
            def _(j16):
                for jj in range(16):
                    j = j16 * 16 + jj
                    for cc in range(4):
                        packb[pl.ds(j * 64 + cc * 16, 16)] = (
                            gb[j, pl.ds(cc * 16, 16)])

            pltpu.sync_copy(packb, out_hbm.at[pl.ds(base * 64, 8192)])

        gather_pack(srcb, gsa, sga, gsrc_hbm)
        gather_pack(dstb, gsd, sgd, gdst_hbm)

        for h in range(8):
            pltpu.make_async_copy(alpb.at[h],
                                  alphaT_hbm.at[h, pl.ds(base, 128)],
                                  sem_w).wait()

    row0 = c * RSC + s * 78

    @pl.loop(0, 78)
    def _(k):
        chunk(row0 + k)

    @pl.when(s < 2)
    def _():
        chunk(c * RSC + 1248 + s)


def _kernel_c(src, dst, emb2, exT, recpl):
    return pl.kernel(
        _kc_body,
        out_type=[jax.ShapeDtypeStruct((E * 64,), _f32),
                  jax.ShapeDtypeStruct((E * 64,), _f32),
                  jax.ShapeDtypeStruct((H, E), _f32)],
        mesh=plsc.VectorSubcoreMesh(core_axis_name="c", subcore_axis_name="s"),
        compiler_params=pltpu.CompilerParams(needs_layout_passes=False),
        scratch_types=[
            pltpu.VMEM((128,), _i32),
            pltpu.VMEM((128,), _i32),
            pltpu.VMEM((128, 128), _f32),
            pltpu.VMEM((128, 128), _f32),
            pltpu.VMEM((8192,), _f32),
            pltpu.VMEM((8, 128), _f32),
            pltpu.VMEM((8, 128), _f32),
            pltpu.SemaphoreType.DMA,
            pltpu.SemaphoreType.DMA,
            pltpu.SemaphoreType.DMA,
            pltpu.VMEM((8 * NP,), _f32),
        ],
    )(src, dst, emb2, exT, recpl.reshape(8 * NP))


def _prep_tc(xp, neW, neb, skW, skb):
    def body(x_ref, nw_ref, nb_ref, sw_ref, sb_ref, z0_ref, skp_ref):
        xb = x_ref[...]
        p = pl.program_id(0)
        z0_ref[...] = jnp.dot(xb, nw_ref[...],
                              preferred_element_type=_f32) + nb_ref[...]
        skp_ref[0] = jnp.dot(xb, sw_ref[...],
                             preferred_element_type=_f32) + sb_ref[pl.ds(p, 1)]

    return pl.pallas_call(
        body,
        grid=(4, NBLK),
        in_specs=[pl.BlockSpec((BLK, 3), lambda p, i: (i, 0)),
                  pl.BlockSpec((3, 64), lambda p, i: (0, 0)),
                  pl.BlockSpec((1, 64), lambda p, i: (0, 0)),
                  pl.BlockSpec((3, 128), lambda p, i: (0, p)),
                  pl.BlockSpec((4, 128), lambda p, i: (0, 0))],
        out_specs=[pl.BlockSpec((BLK, 64), lambda p, i: (i, 0)),
                   pl.BlockSpec((1, BLK, 128), lambda p, i: (p, i, 0))],
        out_shape=[jax.ShapeDtypeStruct((NP, 64), _f32),
                   jax.ShapeDtypeStruct((4, NP, 128), _f32)],
    )(xp, neW, neb.reshape(1, 64), skW, skb.reshape(4, 128))


def _estats_tc(e2d):
    def body(e_ref, o_ref):
        v = e_ref[...]
        o_ref[0, :] = jnp.full((128,), jnp.min(v), _f32)
        o_ref[1, :] = jnp.full((128,), jnp.max(v), _f32)
        o_ref[2, :] = jnp.full((128,), jnp.mean(v), _f32)
        o_ref[3, :] = jnp.zeros((128,), _f32)

    return pl.pallas_call(
        body,
        out_shape=jax.ShapeDtypeStruct((4, 128), _f32),
    )(e2d)


def _asd1_tc(z0, ucat):
    def body(z_ref, u_ref, o_ref):
        o_ref[...] = jnp.dot(z_ref[...], u_ref[...],
                             preferred_element_type=_f32)

    return pl.pallas_call(
        body,
        grid=(NBLK,),
        in_specs=[pl.BlockSpec((BLK, 64), lambda i: (i, 0)),
                  pl.BlockSpec((64, 16), lambda i: (0, 0))],
        out_specs=pl.BlockSpec((BLK, 16), lambda i: (i, 0)),
        out_shape=jax.ShapeDtypeStruct((NP, 16), _f32),
    )(z0, ucat)


def _asd_tc(oP, ucat):
    def body(o_ref, u_ref, out_ref):
        part = jnp.dot(o_ref[0], u_ref[...], preferred_element_type=_f32)

        @pl.when(pl.program_id(1) == 0)
        def _():
            out_ref[...] = jnp.zeros_like(out_ref)

        out_ref[...] += part

    return pl.pallas_call(
        body,
        grid=(NBLK, 4),
        in_specs=[pl.BlockSpec((1, BLK, 128), lambda i, p: (p, i, 0)),
                  pl.BlockSpec((128, 16), lambda i, p: (p, 0))],
        out_specs=pl.BlockSpec((BLK, 16), lambda i, p: (i, 0)),
        out_shape=jax.ShapeDtypeStruct((NP, 16), _f32),
    )(oP, ucat)


def _pre1_tc(z0, w):
    def body(z_ref, w_ref, h_ref):
        h_ref[0] = jnp.dot(z_ref[...], w_ref[...],
                           preferred_element_type=_f32)

    return pl.pallas_call(
        body,
        grid=(4, NBLK),
        in_specs=[pl.BlockSpec((BLK, 64), lambda q, i: (i, 0)),
                  pl.BlockSpec((64, 128), lambda q, i: (0, q))],
        out_specs=pl.BlockSpec((1, BLK, 128), lambda q, i: (q, i, 0)),
        out_shape=jax.ShapeDtypeStruct((4, NP, 128), _f32),
    )(z0, w)


def _pre_tc(oP, w):
    def body(o_ref, w_ref, h_ref):
        part = jnp.dot(o_ref[0], w_ref[...], preferred_element_type=_f32)

        @pl.when(pl.program_id(2) == 0)
        def _():
            h_ref[...] = jnp.zeros_like(h_ref)

        h_ref[0] += part

    return pl.pallas_call(
        body,
        grid=(4, NBLK, 4),
        in_specs=[pl.BlockSpec((1, BLK, 128), lambda q, i, p: (p, i, 0)),
                  pl.BlockSpec((128, 128), lambda q, i, p: (p, q))],
        out_specs=pl.BlockSpec((1, BLK, 128), lambda q, i, p: (q, i, 0)),
        out_shape=jax.ShapeDtypeStruct((4, NP, 128), _f32),
    )(oP, w)


def _dens_tc(ascols, denC, consts):
    def body(a_ref, d_ref, c_ref, rec_ref, exl_ref):
        a = a_ref[...]
        l = a[:, :8] + a[:, 8:] + c_ref[0:1, :]
        lr = jnp.maximum(l, 0.2 * l)
        exl = jnp.exp(lr - c_ref[1:2, :])
        exl_ref[...] = exl
        rec_ref[...] = 1.0 / (d_ref[...] + exl + 1e-16)

    return pl.pallas_call(
        body,
        grid=(NBLK,),
        in_specs=[pl.BlockSpec((BLK, 16), lambda i: (i, 0)),
                  pl.BlockSpec((BLK, 8), lambda i: (i, 0)),
                  pl.BlockSpec((2, 8), lambda i: (0, 0))],
        out_specs=[pl.BlockSpec((BLK, 8), lambda i: (i, 0)),
                   pl.BlockSpec((BLK, 8), lambda i: (i, 0))],
        out_shape=[jax.ShapeDtypeStruct((NP, 8), _f32),
                   jax.ShapeDtypeStruct((NP, 8), _f32)],
    )(ascols, denC, consts)


def _post_tc(aggP, hP, recT, exlT, skipP, bP):
    def body(g_ref, h_ref, r_ref, x_ref, s_ref, b_ref, o_ref):
        for p in range(4):
            rec2 = r_ref[:, 2 * p:2 * p + 2]
            exl2 = x_ref[:, 2 * p:2 * p + 2]
            recs = jnp.concatenate(
                [jnp.broadcast_to(rec2[:, 0:1], (BLK, 64)),
                 jnp.broadcast_to(rec2[:, 1:2], (BLK, 64))], axis=1)
            exls = jnp.concatenate(
                [jnp.broadcast_to(exl2[:, 0:1], (BLK, 64)),
                 jnp.broadcast_to(exl2[:, 1:2], (BLK, 64))], axis=1)
            v = (g_ref[p] + h_ref[p] * exls) * recs + b_ref[p:p + 1]
            o_ref[p] = jnp.where(v > 0, v, (jnp.exp(v) - 1.0)) + s_ref[p]

    return pl.pallas_call(
        body,
        grid=(NBLK,),
        in_specs=[pl.BlockSpec((4, BLK, 128), lambda i: (0, i, 0)),
                  pl.BlockSpec((4, BLK, 128), lambda i: (0, i, 0)),
                  pl.BlockSpec((BLK, 8), lambda i: (i, 0)),
                  pl.BlockSpec((BLK, 8), lambda i: (i, 0)),
                  pl.BlockSpec((4, BLK, 128), lambda i: (0, i, 0)),
                  pl.BlockSpec((4, 128), lambda i: (0, 0))],
        out_specs=pl.BlockSpec((4, BLK, 128), lambda i: (0, i, 0)),
        out_shape=jax.ShapeDtypeStruct((4, NP, 128), _f32),
    )(aggP, hP, recT, exlT, skipP, bP)


def _post3_tc(aggP, hP, recT, exlT, b3, linW, linb):
    def body(g_ref, h_ref, r_ref, x_ref, b_ref, lw_ref, lb_ref,
             emb_ref, xo_ref):
        ssum = jnp.zeros((BLK, 64), _f32)
        for p in range(4):
            for half in range(2):
                hh = 2 * p + half
                seg = g_ref[p, :, 64 * half:64 * half + 64]
                hseg = h_ref[p, :, 64 * half:64 * half + 64]
                ssum += (seg + hseg * x_ref[:, hh:hh + 1]) * r_ref[:, hh:hh + 1]
        o = ssum * 0.125 + b_ref[...]
        embv = jnp.where(o > 0, o, (jnp.exp(o) - 1.0))
        emb_ref[...] = jnp.concatenate(
            [embv, jnp.zeros((BLK, 64), _f32)], axis=1)
        xo_ref[...] = jnp.dot(embv, lw_ref[...],
                              preferred_element_type=_f32) + lb_ref[...]

    return pl.pallas_call(
        body,
        grid=(NBLK,),
        in_specs=[pl.BlockSpec((4, BLK, 128), lambda i: (0, i, 0)),
                  pl.BlockSpec((4, BLK, 128), lambda i: (0, i, 0)),
                  pl.BlockSpec((BLK, 8), lambda i: (i, 0)),
                  pl.BlockSpec((BLK, 8), lambda i: (i, 0)),
                  pl.BlockSpec((1, 64), lambda i: (0, 0)),
                  pl.BlockSpec((64, 2), lambda i: (0, 0)),
                  pl.BlockSpec((1, 2), lambda i: (0, 0))],
        out_specs=[pl.BlockSpec((BLK, 128), lambda i: (i, 0)),
                   pl.BlockSpec((BLK, 2), lambda i: (i, 0))],
        out_shape=[jax.ShapeDtypeStruct((NP, 128), _f32),
                   jax.ShapeDtypeStruct((NP, 2), _f32)],
    )(aggP, hP, recT, exlT, b3, linW, linb)


def _mlp_tc(gsrc, gdst, alpha, a1, a2, a3, b1, w2, b2):
    eblk = 2000

    def body(g1_ref, g2_ref, al_ref, a1_ref, a2_ref, a3_ref, b1_ref,
             w2_ref, b2_ref, o_ref):
        hid = (jnp.dot(g1_ref[...], a1_ref[...], preferred_element_type=_f32)
               + jnp.dot(al_ref[...], a2_ref[...], preferred_element_type=_f32)
               + jnp.dot(g2_ref[...], a3_ref[...], preferred_element_type=_f32)
               + b1_ref[...])
        hid = jnp.maximum(hid, 0.0)
        o_ref[...] = jnp.dot(hid, w2_ref[...],
                             preferred_element_type=_f32) + b2_ref[...]

    return pl.pallas_call(
        body,
        grid=(E // eblk,),
        in_specs=[pl.BlockSpec((eblk, 64), lambda i: (i, 0)),
                  pl.BlockSpec((eblk, 64), lambda i: (i, 0)),
                  pl.BlockSpec((eblk, 8), lambda i: (i, 0)),
                  pl.BlockSpec((64, 256), lambda i: (0, 0)),
                  pl.BlockSpec((8, 256), lambda i: (0, 0)),
                  pl.BlockSpec((64, 256), lambda i: (0, 0)),
                  pl.BlockSpec((1, 256), lambda i: (0, 0)),
                  pl.BlockSpec((256, 2), lambda i: (0, 0)),
                  pl.BlockSpec((1, 2), lambda i: (0, 0))],
        out_specs=pl.BlockSpec((eblk, 2), lambda i: (i, 0)),
        out_shape=jax.ShapeDtypeStruct((E, 2), _f32),
    )(gsrc, gdst, alpha, a1, a2, a3, b1, w2, b2)


def kernel(x, edge_index, edge_attr, return_attention_weights, params):
    p = params
    src = edge_index[0]
    dst = edge_index[1]
    eattr = edge_attr[:, 0]
    xp = jnp.pad(x, ((0, NP - N), (0, 0)))
    zrow = jnp.zeros((64, 128), _f32)

    est = _estats_tc(eattr.reshape(ROWS, 128))
    emin, emax, emean = est[0, 0], est[1, 0], est[2, 0]

    z0, skipP = _prep_tc(xp, p["ne_W"], p["ne_b"], p["skip_W"], p["skip_b"])

    def layer(feats, pp, first):
        w = pp["W"]
        ind = w.shape[0]
        us = (w.reshape(ind, H, C) * pp["att_src"][None]).sum(-1)
        ud = (w.reshape(ind, H, C) * pp["att_dst"][None]).sum(-1)
        ucat = jnp.concatenate([us, ud], axis=1)
        ve = (pp["W_e"].reshape(64, H, C) * pp["att_e"][None]).sum(-1)
        mvec = (p["ee_W"] @ ve)[0]
        cvec = p["ee_b"] @ ve
        ael = emean * mvec + cvec

        if first:
            hP = _pre1_tc(feats, w)
            ascols = _asd1_tc(feats, ucat)
        else:
            hP = _pre_tc(feats, w)
            ascols = _asd_tc(feats, ucat)

        asrc = ascols[:N, :8]
        adst = ascols[:N, 8:]
        ae_max = jnp.maximum(jnp.where(mvec > 0, mvec * emax, mvec * emin)
                             + cvec, ael)
        b = jnp.max(asrc, axis=0) + jnp.max(adst, axis=0) + ae_max
        bl = jnp.where(b >= 0.0, b, 0.2 * b)

        psplat = jnp.stack([
            jnp.broadcast_to(mvec[:, None], (8, 16)),
            jnp.broadcast_to(cvec[:, None], (8, 16)),
            jnp.broadcast_to(bl[:, None], (8, 16))])

        asrcT = jnp.copy(ascols[:, :8].T)
        adstT = jnp.copy(ascols[:, 8:].T)
        exT, den2 = _kernel_a(src, dst, eattr, asrcT, adstT, psplat)

        den = (den2[0] + den2[1]).reshape(8, NP)
        denC = jnp.copy(den.T)
        consts = jnp.stack([ael, bl])
        recT, exlT = _dens_tc(ascols, denC, consts)

        hflat = hP.reshape(4 * NP, 128)
        aggP = _kernel_b(src, dst, exT, hflat, zrow)
        return hP, aggP, recT, exlT, exT

    hP, aggP, recT, exlT, _ = layer(z0, p["c1"], True)
    b1p = p["c1"]["b"].reshape(4, 128)
    oP = _post_tc(aggP, hP, recT, exlT, skipP, b1p)

    hP, aggP, recT, exlT, _ = layer(oP, p["c2"], False)
    b2p = p["c2"]["b"].reshape(4, 128)
    oP = _post_tc(aggP, hP, recT, exlT, skipP, b2p)

    hP, aggP, recT, exlT, exT3 = layer(oP, p["c3"], False)
    emb, xoutp = _post3_tc(aggP, hP, recT, exlT,
                           p["c3"]["b"].reshape(1, 64), p["lin_W"],
                           p["lin_b"].reshape(1, 2))

    rec3pl = jnp.copy(recT.T)
    gsrcf, gdstf, alphaT = _kernel_c(src, dst, emb, exT3, rec3pl)
    gsrc = gsrcf.reshape(E, 64)
    gdst = gdstf.reshape(E, 64)
    alpha = alphaT.T

    w1 = p["mlp_W1"]
    edge_out = _mlp_tc(gsrc, gdst, alpha,
                       w1[:64], w1[64:72], w1[72:],
                       p["mlp_b1"].reshape(1, 256), p["mlp_W2"],
                       p["mlp_b2"].reshape(1, 2))
    return xoutp[:N], edge_out
